# Initial kernel scaffold; baseline (speedup 1.0000x reference)
import functools

import jax
import jax.numpy as jnp
from jax.experimental import pallas as pl
from jax.experimental.pallas import tpu as pltpu

_EPS = 1e-5
_BF = jnp.bfloat16
_F32 = jnp.float32

_PLAN = {56: (58, 64, 1), 28: (30, 32, 4), 14: (16, 16, 8), 7: (10, 9, 8)}


def _ru(x, m):
    return ((x + m - 1) // m) * m


def _bn_scale_shift(cb, cout_p):
    s = cb["gamma"] * jax.lax.rsqrt(cb["var"] + _EPS)
    sh = cb["b"] * s + cb["beta"] - cb["mean"] * s
    cout = s.shape[0]
    if cout_p != cout:
        s = jnp.pad(s, (0, cout_p - cout))
        sh = jnp.pad(sh, (0, cout_p - cout))
    return (s.reshape(1, cout_p).astype(_F32),
            sh.reshape(1, cout_p).astype(_F32))


def _w_1x1(w, cin_p, cout_p):
    cout, cin = w.shape[0], w.shape[1]
    wm = jnp.transpose(w[:, :, 0, 0])
    wm = jnp.pad(wm, ((0, cin_p - cin), (0, cout_p - cout)))
    return wm.astype(_BF)


def _w_3x3(w, cin_p, cout_p):
    cout, cin = w.shape[0], w.shape[1]
    wt = jnp.transpose(w, (2, 3, 1, 0))
    wt = jnp.pad(wt, ((0, 0), (0, 0), (0, cin_p - cin), (0, cout_p - cout)))
    return wt.reshape(9 * cin_p, cout_p).astype(_BF)


def _interior(mstart, bs, S, RW, H, W):
    m = jax.lax.broadcasted_iota(jnp.int32, (bs, 1), 0) + mstart
    p = m % S
    r = p // RW
    c = p % RW
    ok = (r >= 1) & (r <= H) & (c >= 1) & (c <= W)
    return ok.astype(_F32)


def _bneck_kernel(*refs, has_ds, S, RW, H, W, C1, PAD, BS):
    if has_ds:
        (x_ref, w1r, s1r, h1r, w2r, s2r, h2r, w3r, s3r, h3r,
         wdr, sdr, hdr, o_ref, y1s) = refs
    else:
        (x_ref, w1r, s1r, h1r, w2r, s2r, h2r, w3r, s3r, h3r,
         o_ref, y1s) = refs

    msk = _interior(0, BS, S, RW, H, W)
    x = x_ref[...]

    a1 = jnp.dot(x, w1r[...], preferred_element_type=_F32)
    y1 = jnp.maximum(a1 * s1r[...] + h1r[...], 0.0) * msk
    y1s[0:PAD, :] = jnp.zeros((PAD, C1), _BF)
    y1s[PAD + BS:, :] = jnp.zeros((PAD, C1), _BF)
    y1s[PAD:PAD + BS, :] = y1.astype(_BF)

    acc = jnp.zeros((BS, C1), _F32)
    for a in range(3):
        for b in range(3):
            off = PAD + (a - 1) * RW + (b - 1)
            k = a * 3 + b
            acc = acc + jnp.dot(y1s[off:off + BS, :],
                                w2r[k * C1:(k + 1) * C1, :],
                                preferred_element_type=_F32)
    y2 = (jnp.maximum(acc * s2r[...] + h2r[...], 0.0) * msk).astype(_BF)

    a3 = jnp.dot(y2, w3r[...], preferred_element_type=_F32) * s3r[...] + h3r[...]
    if has_ds:
        ident = (jnp.dot(x, wdr[...], preferred_element_type=_F32)
                 * sdr[...] + hdr[...])
    else:
        ident = x.astype(_F32)
    o_ref[...] = (jnp.maximum(a3 + ident, 0.0) * msk).astype(_BF)


def _bottleneck(x, p1, p2, p3, pds, H, W, cin_p, c1_p, cout_p, N):
    rows, RW, B = _PLAN[H]
    S = rows * RW
    BS = B * S
    PAD = _ru(RW + 1, 8)

    w1 = _w_1x1(p1["w"], cin_p, c1_p)
    s1, h1 = _bn_scale_shift(p1, c1_p)
    w2 = _w_3x3(p2["w"], c1_p, c1_p)
    s2, h2 = _bn_scale_shift(p2, c1_p)
    w3 = _w_1x1(p3["w"], c1_p, cout_p)
    s3, h3 = _bn_scale_shift(p3, cout_p)

    args = [x, w1, s1, h1, w2, s2, h2, w3, s3, h3]
    in_specs = [
        pl.BlockSpec((BS, cin_p), lambda i: (i, 0)),
        pl.BlockSpec((cin_p, c1_p), lambda i: (0, 0)),
        pl.BlockSpec((1, c1_p), lambda i: (0, 0)),
        pl.BlockSpec((1, c1_p), lambda i: (0, 0)),
        pl.BlockSpec((9 * c1_p, c1_p), lambda i: (0, 0)),
        pl.BlockSpec((1, c1_p), lambda i: (0, 0)),
        pl.BlockSpec((1, c1_p), lambda i: (0, 0)),
        pl.BlockSpec((c1_p, cout_p), lambda i: (0, 0)),
        pl.BlockSpec((1, cout_p), lambda i: (0, 0)),
        pl.BlockSpec((1, cout_p), lambda i: (0, 0)),
    ]
    if pds is not None:
        wds = _w_1x1(pds["w"], cin_p, cout_p)
        sds, hds = _bn_scale_shift(pds, cout_p)
        args += [wds, sds, hds]
        in_specs += [
            pl.BlockSpec((cin_p, cout_p), lambda i: (0, 0)),
            pl.BlockSpec((1, cout_p), lambda i: (0, 0)),
            pl.BlockSpec((1, cout_p), lambda i: (0, 0)),
        ]

    return pl.pallas_call(
        functools.partial(_bneck_kernel, has_ds=pds is not None, S=S, RW=RW,
                          H=H, W=W, C1=c1_p, PAD=PAD, BS=BS),
        out_shape=jax.ShapeDtypeStruct((N * S, cout_p), _BF),
        grid=(N // B,),
        in_specs=in_specs,
        out_specs=pl.BlockSpec((BS, cout_p), lambda i: (i, 0)),
        scratch_shapes=[pltpu.VMEM((PAD + BS + PAD, c1_p), _BF)],
        compiler_params=pltpu.CompilerParams(
            dimension_semantics=("arbitrary",),
            vmem_limit_bytes=60 * 1024 * 1024),
    )(*args)


def _gemm_kernel(x_ref, w_ref, s_ref, h_ref, o_ref, *, relu, mp, tm):
    y = jnp.dot(x_ref[...], w_ref[...], preferred_element_type=_F32)
    y = y * s_ref[...] + h_ref[...]
    if relu:
        y = jnp.maximum(y, 0.0)
    if mp is not None:
        S, RW, H, W = mp
        y = y * _interior(pl.program_id(0) * tm, tm, S, RW, H, W)
    o_ref[...] = y.astype(o_ref.dtype)


def _gemm_res_kernel(x_ref, w_ref, s_ref, h_ref, r_ref, o_ref, *, relu, mp, tm):
    y = jnp.dot(x_ref[...], w_ref[...], preferred_element_type=_F32)
    y = y * s_ref[...] + h_ref[...] + r_ref[...].astype(_F32)
    if relu:
        y = jnp.maximum(y, 0.0)
    if mp is not None:
        S, RW, H, W = mp
        y = y * _interior(pl.program_id(0) * tm, tm, S, RW, H, W)
    o_ref[...] = y.astype(o_ref.dtype)


def _gemm(x, wm, s, sh, residual=None, relu=True, mp=None):
    M, K = x.shape
    N = wm.shape[1]
    tm = 512
    while M % tm:
        tm -= 16
    tn = 256 if N % 256 == 0 else N

    args = [x, wm, s, sh]
    in_specs = [
        pl.BlockSpec((tm, K), lambda i, j: (i, 0)),
        pl.BlockSpec((K, tn), lambda i, j: (0, j)),
        pl.BlockSpec((1, tn), lambda i, j: (0, j)),
        pl.BlockSpec((1, tn), lambda i, j: (0, j)),
    ]
    if residual is not None:
        args.append(residual)
        in_specs.append(pl.BlockSpec((tm, tn), lambda i, j: (i, j)))
        body = functools.partial(_gemm_res_kernel, relu=relu, mp=mp, tm=tm)
    else:
        body = functools.partial(_gemm_kernel, relu=relu, mp=mp, tm=tm)

    return pl.pallas_call(
        body,
        out_shape=jax.ShapeDtypeStruct((M, N), _BF),
        grid=(M // tm, N // tn),
        in_specs=in_specs,
        out_specs=pl.BlockSpec((tm, tn), lambda i, j: (i, j)),
        compiler_params=pltpu.CompilerParams(
            dimension_semantics=("arbitrary", "arbitrary"),
            vmem_limit_bytes=60 * 1024 * 1024),
    )(*args)


_S0 = 116 * 116
_RW0 = 116
_M0 = 112 * _RW0


def _stem_kernel(x_ref, w_ref, s_ref, h_ref, o_ref):
    f = x_ref[...]
    n3 = _S0 - 3
    g = jnp.concatenate(
        [f[0:n3], f[1:1 + n3], f[2:2 + n3], f[3:3 + n3]], axis=1)
    h4 = jnp.concatenate(
        [g[0:_M0], g[_RW0:_RW0 + _M0],
         g[2 * _RW0:2 * _RW0 + _M0], g[3 * _RW0:3 * _RW0 + _M0]],
        axis=1)
    acc = jnp.dot(h4, w_ref[...], preferred_element_type=_F32)
    o_ref[...] = (jnp.maximum(acc * s_ref[...] + h_ref[...], 0.0)).astype(_BF)


def _stem(x, cb, N):
    xn = jnp.transpose(x, (0, 2, 3, 1))
    xp = jnp.pad(xn, ((0, 0), (3, 3), (3, 3), (0, 0)))
    x2 = xp.reshape(N, 115, 2, 115, 2, 3).transpose(0, 1, 3, 2, 4, 5)
    x2 = x2.reshape(N, 115, 115, 12)
    x2 = jnp.pad(x2, ((0, 0), (0, 1), (0, 1), (0, 4))).astype(_BF)
    x2 = x2.reshape(N * _S0, 16)

    wt = cb["w"]
    wp = jnp.pad(wt, ((0, 0), (0, 0), (0, 1), (0, 1)))
    wp = wp.reshape(64, 3, 4, 2, 4, 2).transpose(2, 4, 3, 5, 1, 0)
    wp = wp.reshape(4, 4, 12, 64)
    wp = jnp.pad(wp, ((0, 0), (0, 0), (0, 4), (0, 0)))
    wst = wp.reshape(256, 64).astype(_BF)
    sst, hst = _bn_scale_shift(cb, 64)

    return pl.pallas_call(
        _stem_kernel,
        out_shape=jax.ShapeDtypeStruct((N * _M0, 64), _BF),
        grid=(N,),
        in_specs=[
            pl.BlockSpec((_S0, 16), lambda i: (i, 0)),
            pl.BlockSpec((256, 64), lambda i: (0, 0)),
            pl.BlockSpec((1, 64), lambda i: (0, 0)),
            pl.BlockSpec((1, 64), lambda i: (0, 0)),
        ],
        out_specs=pl.BlockSpec((_M0, 64), lambda i: (i, 0)),
        compiler_params=pltpu.CompilerParams(
            dimension_semantics=("arbitrary",),
            vmem_limit_bytes=60 * 1024 * 1024),
    )(x2, wst, sst, hst)


def _transition(x, bp, HA, WA, cin_p, c1_p, cout_p, N):
    rowsA, RWA, _ = _PLAN[HA]
    SA = rowsA * RWA
    HB, WB = HA // 2, WA // 2
    rowsB, RWB, _ = _PLAN[HB]
    SB = rowsB * RWB

    w1 = _w_1x1(bp["conv1"]["w"], cin_p, c1_p)
    s1, h1 = _bn_scale_shift(bp["conv1"], c1_p)
    y1 = _gemm(x, w1, s1, h1, relu=True, mp=(SA, RWA, HA, WA))

    y14 = y1.reshape(N, rowsA, RWA, c1_p)
    cols = []
    for i in range(3):
        for j in range(3):
            cols.append(y14[:, i:i + 2 * HB:2, j:j + 2 * WB:2, :])
    patches = jnp.stack(cols, axis=3).reshape(N * HB * WB, 9 * c1_p)

    w2 = _w_3x3(bp["conv2"]["w"], c1_p, c1_p)
    s2, h2 = _bn_scale_shift(bp["conv2"], c1_p)
    y2 = _gemm(patches, w2, s2, h2, relu=True, mp=None)
    y2p = jnp.pad(y2.reshape(N, HB, WB, c1_p),
                  ((0, 0), (1, rowsB - HB - 1), (1, RWB - WB - 1), (0, 0)))
    y2p = y2p.reshape(N * SB, c1_p)

    sub = x.reshape(N, rowsA, RWA, cin_p)[:, 1:1 + 2 * HB:2, 1:1 + 2 * WB:2, :]
    subp = jnp.pad(sub,
                   ((0, 0), (1, rowsB - HB - 1), (1, RWB - WB - 1), (0, 0)))
    subp = subp.reshape(N * SB, cin_p)
    wd = _w_1x1(bp["ds"]["w"], cin_p, cout_p)
    sd, hd = _bn_scale_shift(bp["ds"], cout_p)
    ident = _gemm(subp, wd, sd, hd, relu=False, mp=None)

    w3 = _w_1x1(bp["conv3"]["w"], c1_p, cout_p)
    s3, h3 = _bn_scale_shift(bp["conv3"], cout_p)
    return _gemm(y2p, w3, s3, h3, residual=ident, relu=True,
                 mp=(SB, RWB, HB, WB))


def _fc_kernel(a_ref, x_ref, w_ref, b_ref, o_ref, *, inv_s):
    t = jnp.dot(a_ref[...], x_ref[...], preferred_element_type=_F32)
    xm = (t * inv_s).astype(_BF)
    o_ref[...] = jnp.dot(xm, w_ref[...], preferred_element_type=_F32) + b_ref[...]


def _avgpool_fc(x, fc_w, fc_b, S, H, W, N):
    C = x.shape[1]
    nc = fc_w.shape[0]
    ncp = _ru(nc, 128)
    amat = jnp.repeat(jnp.eye(N, dtype=_BF), S, axis=1)
    wm = jnp.pad(jnp.transpose(fc_w), ((0, 0), (0, ncp - nc))).astype(_BF)
    bv = jnp.pad(fc_b, (0, ncp - nc)).reshape(1, ncp).astype(_F32)
    out = pl.pallas_call(
        functools.partial(_fc_kernel, inv_s=1.0 / float(H * W)),
        out_shape=jax.ShapeDtypeStruct((N, ncp), _F32),
        grid=(1,),
        in_specs=[
            pl.BlockSpec((N, N * S), lambda i: (0, 0)),
            pl.BlockSpec((N * S, C), lambda i: (0, 0)),
            pl.BlockSpec((C, ncp), lambda i: (0, 0)),
            pl.BlockSpec((1, ncp), lambda i: (0, 0)),
        ],
        out_specs=pl.BlockSpec((N, ncp), lambda i: (0, 0)),
        compiler_params=pltpu.CompilerParams(
            dimension_semantics=("arbitrary",),
            vmem_limit_bytes=60 * 1024 * 1024),
    )(amat, x, wm, bv)
    return out[:, :nc]


_SUF = ("w", "b", "gamma", "beta", "mean", "var")


def kernel(*args):
    x = args[0]
    stem = dict(zip(_SUF, args[1:7]))
    idx = 7
    layers = []
    for nblocks in (3, 4, 6, 3):
        blocks = []
        for b in range(nblocks):
            bp = {}
            for cname in ("conv1", "conv2", "conv3"):
                bp[cname] = dict(zip(_SUF, args[idx:idx + 6]))
                idx += 6
            if b == 0:
                bp["ds"] = dict(zip(_SUF, args[idx:idx + 6]))
                idx += 6
            blocks.append(bp)
        layers.append(blocks)
    fc_w, fc_b = args[idx], args[idx + 1]

    N = x.shape[0]

    y = _stem(x, stem, N)
    y4 = y.reshape(N, 112, 116, 64)
    pool = jax.lax.reduce_window(
        y4, jnp.array(-jnp.inf, _BF), jax.lax.max,
        window_dimensions=(1, 3, 3, 1), window_strides=(1, 2, 2, 1),
        padding=((0, 0), (1, 1), (1, 1), (0, 0)))[:, :, :56, :]
    rows1, RW1, _ = _PLAN[56]
    h = jnp.pad(pool, ((0, 0), (1, rows1 - 57), (1, RW1 - 57), (0, 64)))
    h = h.reshape(N * rows1 * RW1, 128)

    cfg = [(56, 128, 128, 256), (28, 256, 128, 512),
           (14, 512, 256, 1024), (7, 1024, 512, 2048)]
    for li, (H, cin_p, c1_p, cout_p) in enumerate(cfg):
        blocks = layers[li]
        if li == 0:
            h = _bottleneck(h, blocks[0]["conv1"], blocks[0]["conv2"],
                            blocks[0]["conv3"], blocks[0]["ds"],
                            H, H, cin_p, c1_p, cout_p, N)
        else:
            h = _transition(h, blocks[0], H * 2, H * 2, cin_p, c1_p, cout_p, N)
        for bp in blocks[1:]:
            h = _bottleneck(h, bp["conv1"], bp["conv2"], bp["conv3"], None,
                            H, H, cout_p, c1_p, cout_p, N)

    rows4, RW4, _ = _PLAN[7]
    return _avgpool_fc(h, fc_w, fc_b, rows4 * RW4, 7, 7, N)


# speedup vs baseline: 1.5977x; 1.5977x over previous
import functools

import jax
import jax.numpy as jnp
from jax.experimental import pallas as pl
from jax.experimental.pallas import tpu as pltpu

_EPS = 1e-5
_BF = jnp.bfloat16
_F32 = jnp.float32

_PLAN = {56: (58, 64, 1), 28: (30, 32, 4), 14: (16, 16, 8), 7: (10, 9, 8)}


def _ru(x, m):
    return ((x + m - 1) // m) * m


def _bn_scale_shift(cb, cout_p):
    s = cb["gamma"] * jax.lax.rsqrt(cb["var"] + _EPS)
    sh = cb["b"] * s + cb["beta"] - cb["mean"] * s
    cout = s.shape[0]
    if cout_p != cout:
        s = jnp.pad(s, (0, cout_p - cout))
        sh = jnp.pad(sh, (0, cout_p - cout))
    return (s.reshape(1, cout_p).astype(_F32),
            sh.reshape(1, cout_p).astype(_F32))


def _w_1x1(w, cin_p, cout_p):
    cout, cin = w.shape[0], w.shape[1]
    wm = jnp.transpose(w[:, :, 0, 0])
    wm = jnp.pad(wm, ((0, cin_p - cin), (0, cout_p - cout)))
    return wm.astype(_BF)


def _w_3x3(w, cin_p, cout_p):
    cout, cin = w.shape[0], w.shape[1]
    wt = jnp.transpose(w, (2, 3, 1, 0))
    wt = jnp.pad(wt, ((0, 0), (0, 0), (0, cin_p - cin), (0, cout_p - cout)))
    return wt.reshape(9 * cin_p, cout_p).astype(_BF)


def _interior(mstart, bs, S, RW, H, W):
    m = jax.lax.broadcasted_iota(jnp.int32, (bs, 1), 0) + mstart
    p = m % S
    r = p // RW
    c = p % RW
    ok = (r >= 1) & (r <= H) & (c >= 1) & (c <= W)
    return ok.astype(_F32)


def _bneck_kernel(*refs, has_ds, S, RW, H, W, C1, PAD, BS):
    if has_ds:
        (x_ref, w1r, s1r, h1r, w2r, s2r, h2r, w3r, s3r, h3r,
         wdr, sdr, hdr, o_ref, y1s) = refs
    else:
        (x_ref, w1r, s1r, h1r, w2r, s2r, h2r, w3r, s3r, h3r,
         o_ref, y1s) = refs

    msk = _interior(0, BS, S, RW, H, W)
    x = x_ref[...]

    a1 = jnp.dot(x, w1r[...], preferred_element_type=_F32)
    y1 = jnp.maximum(a1 * s1r[...] + h1r[...], 0.0) * msk
    y1s[0:PAD, :] = jnp.zeros((PAD, C1), _BF)
    y1s[PAD + BS:, :] = jnp.zeros((PAD, C1), _BF)
    y1s[PAD:PAD + BS, :] = y1.astype(_BF)

    acc = jnp.zeros((BS, C1), _F32)
    for a in range(3):
        for b in range(3):
            off = PAD + (a - 1) * RW + (b - 1)
            k = a * 3 + b
            acc = acc + jnp.dot(y1s[off:off + BS, :],
                                w2r[k * C1:(k + 1) * C1, :],
                                preferred_element_type=_F32)
    y2 = (jnp.maximum(acc * s2r[...] + h2r[...], 0.0) * msk).astype(_BF)

    a3 = jnp.dot(y2, w3r[...], preferred_element_type=_F32) * s3r[...] + h3r[...]
    if has_ds:
        ident = (jnp.dot(x, wdr[...], preferred_element_type=_F32)
                 * sdr[...] + hdr[...])
    else:
        ident = x.astype(_F32)
    o_ref[...] = (jnp.maximum(a3 + ident, 0.0) * msk).astype(_BF)


def _bottleneck(x, p1, p2, p3, pds, H, W, cin_p, c1_p, cout_p, N):
    rows, RW, B = _PLAN[H]
    S = rows * RW
    BS = B * S
    PAD = _ru(RW + 1, 8)

    w1 = _w_1x1(p1["w"], cin_p, c1_p)
    s1, h1 = _bn_scale_shift(p1, c1_p)
    w2 = _w_3x3(p2["w"], c1_p, c1_p)
    s2, h2 = _bn_scale_shift(p2, c1_p)
    w3 = _w_1x1(p3["w"], c1_p, cout_p)
    s3, h3 = _bn_scale_shift(p3, cout_p)

    args = [x, w1, s1, h1, w2, s2, h2, w3, s3, h3]
    in_specs = [
        pl.BlockSpec((BS, cin_p), lambda i: (i, 0)),
        pl.BlockSpec((cin_p, c1_p), lambda i: (0, 0)),
        pl.BlockSpec((1, c1_p), lambda i: (0, 0)),
        pl.BlockSpec((1, c1_p), lambda i: (0, 0)),
        pl.BlockSpec((9 * c1_p, c1_p), lambda i: (0, 0)),
        pl.BlockSpec((1, c1_p), lambda i: (0, 0)),
        pl.BlockSpec((1, c1_p), lambda i: (0, 0)),
        pl.BlockSpec((c1_p, cout_p), lambda i: (0, 0)),
        pl.BlockSpec((1, cout_p), lambda i: (0, 0)),
        pl.BlockSpec((1, cout_p), lambda i: (0, 0)),
    ]
    if pds is not None:
        wds = _w_1x1(pds["w"], cin_p, cout_p)
        sds, hds = _bn_scale_shift(pds, cout_p)
        args += [wds, sds, hds]
        in_specs += [
            pl.BlockSpec((cin_p, cout_p), lambda i: (0, 0)),
            pl.BlockSpec((1, cout_p), lambda i: (0, 0)),
            pl.BlockSpec((1, cout_p), lambda i: (0, 0)),
        ]

    return pl.pallas_call(
        functools.partial(_bneck_kernel, has_ds=pds is not None, S=S, RW=RW,
                          H=H, W=W, C1=c1_p, PAD=PAD, BS=BS),
        out_shape=jax.ShapeDtypeStruct((N * S, cout_p), _BF),
        grid=(N // B,),
        in_specs=in_specs,
        out_specs=pl.BlockSpec((BS, cout_p), lambda i: (i, 0)),
        scratch_shapes=[pltpu.VMEM((PAD + BS + PAD, c1_p), _BF)],
        compiler_params=pltpu.CompilerParams(
            dimension_semantics=("arbitrary",),
            vmem_limit_bytes=60 * 1024 * 1024),
    )(*args)


def _gemm_kernel(x_ref, w_ref, s_ref, h_ref, o_ref, *, relu, mp, tm):
    y = jnp.dot(x_ref[...], w_ref[...], preferred_element_type=_F32)
    y = y * s_ref[...] + h_ref[...]
    if relu:
        y = jnp.maximum(y, 0.0)
    if mp is not None:
        S, RW, H, W = mp
        y = y * _interior(pl.program_id(0) * tm, tm, S, RW, H, W)
    o_ref[...] = y.astype(o_ref.dtype)


def _gemm_res_kernel(x_ref, w_ref, s_ref, h_ref, r_ref, o_ref, *, relu, mp, tm):
    y = jnp.dot(x_ref[...], w_ref[...], preferred_element_type=_F32)
    y = y * s_ref[...] + h_ref[...] + r_ref[...].astype(_F32)
    if relu:
        y = jnp.maximum(y, 0.0)
    if mp is not None:
        S, RW, H, W = mp
        y = y * _interior(pl.program_id(0) * tm, tm, S, RW, H, W)
    o_ref[...] = y.astype(o_ref.dtype)


def _gemm(x, wm, s, sh, residual=None, relu=True, mp=None):
    M, K = x.shape
    N = wm.shape[1]
    tm = 512
    while tm > 0 and M % tm:
        tm -= 16
    if tm == 0:
        tm = M
    tn = 256 if N % 256 == 0 else N

    args = [x, wm, s, sh]
    in_specs = [
        pl.BlockSpec((tm, K), lambda i, j: (i, 0)),
        pl.BlockSpec((K, tn), lambda i, j: (0, j)),
        pl.BlockSpec((1, tn), lambda i, j: (0, j)),
        pl.BlockSpec((1, tn), lambda i, j: (0, j)),
    ]
    if residual is not None:
        args.append(residual)
        in_specs.append(pl.BlockSpec((tm, tn), lambda i, j: (i, j)))
        body = functools.partial(_gemm_res_kernel, relu=relu, mp=mp, tm=tm)
    else:
        body = functools.partial(_gemm_kernel, relu=relu, mp=mp, tm=tm)

    return pl.pallas_call(
        body,
        out_shape=jax.ShapeDtypeStruct((M, N), _BF),
        grid=(M // tm, N // tn),
        in_specs=in_specs,
        out_specs=pl.BlockSpec((tm, tn), lambda i, j: (i, j)),
        compiler_params=pltpu.CompilerParams(
            dimension_semantics=("arbitrary", "arbitrary"),
            vmem_limit_bytes=60 * 1024 * 1024),
    )(*args)


_S0 = 116 * 116
_RW0 = 116
_M0 = 112 * _RW0


def _stem_kernel(x_ref, w_ref, s_ref, h_ref, o_ref):
    f = x_ref[...]
    n3 = _S0 - 3
    g = jnp.concatenate(
        [f[0:n3], f[1:1 + n3], f[2:2 + n3], f[3:3 + n3]], axis=1)
    h4 = jnp.concatenate(
        [g[0:_M0], g[_RW0:_RW0 + _M0],
         g[2 * _RW0:2 * _RW0 + _M0], g[3 * _RW0:3 * _RW0 + _M0]],
        axis=1)
    acc = jnp.dot(h4, w_ref[...], preferred_element_type=_F32)
    o_ref[...] = (jnp.maximum(acc * s_ref[...] + h_ref[...], 0.0)).astype(_BF)


def _stem(x, cb, N):
    xn = jnp.transpose(x, (0, 2, 3, 1))
    xp = jnp.pad(xn, ((0, 0), (3, 3), (3, 3), (0, 0)))
    x2 = xp.reshape(N, 115, 2, 115, 2, 3).transpose(0, 1, 3, 2, 4, 5)
    x2 = x2.reshape(N, 115, 115, 12)
    x2 = jnp.pad(x2, ((0, 0), (0, 1), (0, 1), (0, 4))).astype(_BF)
    x2 = x2.reshape(N * _S0, 16)

    wt = cb["w"]
    wp = jnp.pad(wt, ((0, 0), (0, 0), (0, 1), (0, 1)))
    wp = wp.reshape(64, 3, 4, 2, 4, 2).transpose(2, 4, 3, 5, 1, 0)
    wp = wp.reshape(4, 4, 12, 64)
    wp = jnp.pad(wp, ((0, 0), (0, 0), (0, 4), (0, 0)))
    wst = wp.reshape(256, 64).astype(_BF)
    sst, hst = _bn_scale_shift(cb, 64)

    return pl.pallas_call(
        _stem_kernel,
        out_shape=jax.ShapeDtypeStruct((N * _M0, 64), _BF),
        grid=(N,),
        in_specs=[
            pl.BlockSpec((_S0, 16), lambda i: (i, 0)),
            pl.BlockSpec((256, 64), lambda i: (0, 0)),
            pl.BlockSpec((1, 64), lambda i: (0, 0)),
            pl.BlockSpec((1, 64), lambda i: (0, 0)),
        ],
        out_specs=pl.BlockSpec((_M0, 64), lambda i: (i, 0)),
        compiler_params=pltpu.CompilerParams(
            dimension_semantics=("arbitrary",),
            vmem_limit_bytes=60 * 1024 * 1024),
    )(x2, wst, sst, hst)


def _transition(x, bp, HA, WA, cin_p, c1_p, cout_p, N):
    rowsA, RWA, _ = _PLAN[HA]
    SA = rowsA * RWA
    HB, WB = HA // 2, WA // 2
    rowsB, RWB, _ = _PLAN[HB]
    SB = rowsB * RWB

    w1 = _w_1x1(bp["conv1"]["w"], cin_p, c1_p)
    s1, h1 = _bn_scale_shift(bp["conv1"], c1_p)
    y1 = _gemm(x, w1, s1, h1, relu=True, mp=(SA, RWA, HA, WA))

    y14 = y1.reshape(N, rowsA, RWA, c1_p)
    cols = []
    for i in range(3):
        for j in range(3):
            cols.append(y14[:, i:i + 2 * HB:2, j:j + 2 * WB:2, :])
    patches = jnp.stack(cols, axis=3).reshape(N * HB * WB, 9 * c1_p)

    w2 = _w_3x3(bp["conv2"]["w"], c1_p, c1_p)
    s2, h2 = _bn_scale_shift(bp["conv2"], c1_p)
    y2 = _gemm(patches, w2, s2, h2, relu=True, mp=None)
    y2p = jnp.pad(y2.reshape(N, HB, WB, c1_p),
                  ((0, 0), (1, rowsB - HB - 1), (1, RWB - WB - 1), (0, 0)))
    y2p = y2p.reshape(N * SB, c1_p)

    sub = x.reshape(N, rowsA, RWA, cin_p)[:, 1:1 + 2 * HB:2, 1:1 + 2 * WB:2, :]
    subp = jnp.pad(sub,
                   ((0, 0), (1, rowsB - HB - 1), (1, RWB - WB - 1), (0, 0)))
    subp = subp.reshape(N * SB, cin_p)
    wd = _w_1x1(bp["ds"]["w"], cin_p, cout_p)
    sd, hd = _bn_scale_shift(bp["ds"], cout_p)
    ident = _gemm(subp, wd, sd, hd, relu=False, mp=None)

    w3 = _w_1x1(bp["conv3"]["w"], c1_p, cout_p)
    s3, h3 = _bn_scale_shift(bp["conv3"], cout_p)
    return _gemm(y2p, w3, s3, h3, residual=ident, relu=True,
                 mp=(SB, RWB, HB, WB))


def _fc_kernel(a_ref, x_ref, w_ref, b_ref, o_ref, *, inv_s):
    t = jnp.dot(a_ref[...], x_ref[...], preferred_element_type=_F32)
    xm = (t * inv_s).astype(_BF)
    o_ref[...] = jnp.dot(xm, w_ref[...], preferred_element_type=_F32) + b_ref[...]


def _avgpool_fc(x, fc_w, fc_b, S, H, W, N):
    C = x.shape[1]
    nc = fc_w.shape[0]
    ncp = _ru(nc, 128)
    amat = jnp.repeat(jnp.eye(N, dtype=_BF), S, axis=1)
    wm = jnp.pad(jnp.transpose(fc_w), ((0, 0), (0, ncp - nc))).astype(_BF)
    bv = jnp.pad(fc_b, (0, ncp - nc)).reshape(1, ncp).astype(_F32)
    out = pl.pallas_call(
        functools.partial(_fc_kernel, inv_s=1.0 / float(H * W)),
        out_shape=jax.ShapeDtypeStruct((N, ncp), _F32),
        grid=(1,),
        in_specs=[
            pl.BlockSpec((N, N * S), lambda i: (0, 0)),
            pl.BlockSpec((N * S, C), lambda i: (0, 0)),
            pl.BlockSpec((C, ncp), lambda i: (0, 0)),
            pl.BlockSpec((1, ncp), lambda i: (0, 0)),
        ],
        out_specs=pl.BlockSpec((N, ncp), lambda i: (0, 0)),
        compiler_params=pltpu.CompilerParams(
            dimension_semantics=("arbitrary",),
            vmem_limit_bytes=60 * 1024 * 1024),
    )(amat, x, wm, bv)
    return out[:, :nc]


_SUF = ("w", "b", "gamma", "beta", "mean", "var")


def kernel(*args):
    x = args[0]
    stem = dict(zip(_SUF, args[1:7]))
    idx = 7
    layers = []
    for nblocks in (3, 4, 6, 3):
        blocks = []
        for b in range(nblocks):
            bp = {}
            for cname in ("conv1", "conv2", "conv3"):
                bp[cname] = dict(zip(_SUF, args[idx:idx + 6]))
                idx += 6
            if b == 0:
                bp["ds"] = dict(zip(_SUF, args[idx:idx + 6]))
                idx += 6
            blocks.append(bp)
        layers.append(blocks)
    fc_w, fc_b = args[idx], args[idx + 1]

    N = x.shape[0]

    y = _stem(x, stem, N)
    y4 = y.reshape(N, 112, 116, 64)
    pool = jax.lax.reduce_window(
        y4, jnp.array(-jnp.inf, _BF), jax.lax.max,
        window_dimensions=(1, 3, 3, 1), window_strides=(1, 2, 2, 1),
        padding=((0, 0), (1, 1), (1, 1), (0, 0)))[:, :, :56, :]
    rows1, RW1, _ = _PLAN[56]
    h = jnp.pad(pool, ((0, 0), (1, rows1 - 57), (1, RW1 - 57), (0, 64)))
    h = h.reshape(N * rows1 * RW1, 128)

    cfg = [(56, 128, 128, 256), (28, 256, 128, 512),
           (14, 512, 256, 1024), (7, 1024, 512, 2048)]
    for li, (H, cin_p, c1_p, cout_p) in enumerate(cfg):
        blocks = layers[li]
        if li == 0:
            h = _bottleneck(h, blocks[0]["conv1"], blocks[0]["conv2"],
                            blocks[0]["conv3"], blocks[0]["ds"],
                            H, H, cin_p, c1_p, cout_p, N)
        else:
            h = _transition(h, blocks[0], H * 2, H * 2, cin_p, c1_p, cout_p, N)
        for bp in blocks[1:]:
            h = _bottleneck(h, bp["conv1"], bp["conv2"], bp["conv3"], None,
                            H, H, cout_p, c1_p, cout_p, N)

    rows4, RW4, _ = _PLAN[7]
    return _avgpool_fc(h, fc_w, fc_b, rows4 * RW4, 7, 7, N)


# speedup vs baseline: 8.2722x; 5.1776x over previous
import functools

import jax
import jax.numpy as jnp
from jax.experimental import pallas as pl
from jax.experimental.pallas import tpu as pltpu

_EPS = 1e-5
_BF = jnp.bfloat16
_F32 = jnp.float32

_PLAN = {56: (58, 64, 1), 28: (30, 32, 4), 14: (16, 16, 8), 7: (10, 9, 8)}


def _ru(x, m):
    return ((x + m - 1) // m) * m


def _bn_scale_shift(cb, cout_p):
    s = cb["gamma"] * jax.lax.rsqrt(cb["var"] + _EPS)
    sh = cb["b"] * s + cb["beta"] - cb["mean"] * s
    cout = s.shape[0]
    if cout_p != cout:
        s = jnp.pad(s, (0, cout_p - cout))
        sh = jnp.pad(sh, (0, cout_p - cout))
    return (s.reshape(1, cout_p).astype(_F32),
            sh.reshape(1, cout_p).astype(_F32))


def _w_1x1(w, cin_p, cout_p):
    cout, cin = w.shape[0], w.shape[1]
    wm = jnp.transpose(w[:, :, 0, 0])
    wm = jnp.pad(wm, ((0, cin_p - cin), (0, cout_p - cout)))
    return wm.astype(_BF)


def _w_3x3(w, cin_p, cout_p):
    cout, cin = w.shape[0], w.shape[1]
    wt = jnp.transpose(w, (2, 3, 1, 0))
    wt = jnp.pad(wt, ((0, 0), (0, 0), (0, cin_p - cin), (0, cout_p - cout)))
    return wt.reshape(9 * cin_p, cout_p).astype(_BF)


def _interior(mstart, bs, S, RW, H, W):
    m = jax.lax.broadcasted_iota(jnp.int32, (bs, 1), 0) + mstart
    p = m % S
    r = p // RW
    c = p % RW
    ok = (r >= 1) & (r <= H) & (c >= 1) & (c <= W)
    return ok.astype(_F32)


def _bneck_kernel(*refs, has_ds, S, RW, H, W, C1, PAD, BS):
    if has_ds:
        (x_ref, w1r, s1r, h1r, w2r, s2r, h2r, w3r, s3r, h3r,
         wdr, sdr, hdr, o_ref, y1s) = refs
    else:
        (x_ref, w1r, s1r, h1r, w2r, s2r, h2r, w3r, s3r, h3r,
         o_ref, y1s) = refs

    msk = _interior(0, BS, S, RW, H, W)
    x = x_ref[...]

    a1 = jnp.dot(x, w1r[...], preferred_element_type=_F32)
    y1 = jnp.maximum(a1 * s1r[...] + h1r[...], 0.0) * msk
    y1s[0:PAD, :] = jnp.zeros((PAD, C1), _BF)
    y1s[PAD + BS:, :] = jnp.zeros((PAD, C1), _BF)
    y1s[PAD:PAD + BS, :] = y1.astype(_BF)

    acc = jnp.zeros((BS, C1), _F32)
    for a in range(3):
        for b in range(3):
            off = PAD + (a - 1) * RW + (b - 1)
            k = a * 3 + b
            acc = acc + jnp.dot(y1s[off:off + BS, :],
                                w2r[k * C1:(k + 1) * C1, :],
                                preferred_element_type=_F32)
    y2 = (jnp.maximum(acc * s2r[...] + h2r[...], 0.0) * msk).astype(_BF)

    a3 = jnp.dot(y2, w3r[...], preferred_element_type=_F32) * s3r[...] + h3r[...]
    if has_ds:
        ident = (jnp.dot(x, wdr[...], preferred_element_type=_F32)
                 * sdr[...] + hdr[...])
    else:
        ident = x.astype(_F32)
    o_ref[...] = (jnp.maximum(a3 + ident, 0.0) * msk).astype(_BF)


def _bottleneck(x, p1, p2, p3, pds, H, W, cin_p, c1_p, cout_p, N):
    rows, RW, B = _PLAN[H]
    S = rows * RW
    BS = B * S
    PAD = _ru(RW + 1, 8)

    w1 = _w_1x1(p1["w"], cin_p, c1_p)
    s1, h1 = _bn_scale_shift(p1, c1_p)
    w2 = _w_3x3(p2["w"], c1_p, c1_p)
    s2, h2 = _bn_scale_shift(p2, c1_p)
    w3 = _w_1x1(p3["w"], c1_p, cout_p)
    s3, h3 = _bn_scale_shift(p3, cout_p)

    args = [x, w1, s1, h1, w2, s2, h2, w3, s3, h3]
    in_specs = [
        pl.BlockSpec((BS, cin_p), lambda i: (i, 0)),
        pl.BlockSpec((cin_p, c1_p), lambda i: (0, 0)),
        pl.BlockSpec((1, c1_p), lambda i: (0, 0)),
        pl.BlockSpec((1, c1_p), lambda i: (0, 0)),
        pl.BlockSpec((9 * c1_p, c1_p), lambda i: (0, 0)),
        pl.BlockSpec((1, c1_p), lambda i: (0, 0)),
        pl.BlockSpec((1, c1_p), lambda i: (0, 0)),
        pl.BlockSpec((c1_p, cout_p), lambda i: (0, 0)),
        pl.BlockSpec((1, cout_p), lambda i: (0, 0)),
        pl.BlockSpec((1, cout_p), lambda i: (0, 0)),
    ]
    if pds is not None:
        wds = _w_1x1(pds["w"], cin_p, cout_p)
        sds, hds = _bn_scale_shift(pds, cout_p)
        args += [wds, sds, hds]
        in_specs += [
            pl.BlockSpec((cin_p, cout_p), lambda i: (0, 0)),
            pl.BlockSpec((1, cout_p), lambda i: (0, 0)),
            pl.BlockSpec((1, cout_p), lambda i: (0, 0)),
        ]

    return pl.pallas_call(
        functools.partial(_bneck_kernel, has_ds=pds is not None, S=S, RW=RW,
                          H=H, W=W, C1=c1_p, PAD=PAD, BS=BS),
        out_shape=jax.ShapeDtypeStruct((N * S, cout_p), _BF),
        grid=(N // B,),
        in_specs=in_specs,
        out_specs=pl.BlockSpec((BS, cout_p), lambda i: (i, 0)),
        scratch_shapes=[pltpu.VMEM((PAD + BS + PAD, c1_p), _BF)],
        compiler_params=pltpu.CompilerParams(
            dimension_semantics=("arbitrary",),
            vmem_limit_bytes=60 * 1024 * 1024),
    )(*args)


def _gemm_kernel(x_ref, w_ref, s_ref, h_ref, o_ref, *, relu, mp, tm):
    y = jnp.dot(x_ref[...], w_ref[...], preferred_element_type=_F32)
    y = y * s_ref[...] + h_ref[...]
    if relu:
        y = jnp.maximum(y, 0.0)
    if mp is not None:
        S, RW, H, W = mp
        y = y * _interior(pl.program_id(0) * tm, tm, S, RW, H, W)
    o_ref[...] = y.astype(o_ref.dtype)


def _gemm_res_kernel(x_ref, w_ref, s_ref, h_ref, r_ref, o_ref, *, relu, mp, tm):
    y = jnp.dot(x_ref[...], w_ref[...], preferred_element_type=_F32)
    y = y * s_ref[...] + h_ref[...] + r_ref[...].astype(_F32)
    if relu:
        y = jnp.maximum(y, 0.0)
    if mp is not None:
        S, RW, H, W = mp
        y = y * _interior(pl.program_id(0) * tm, tm, S, RW, H, W)
    o_ref[...] = y.astype(o_ref.dtype)


def _gemm(x, wm, s, sh, residual=None, relu=True, mp=None):
    M, K = x.shape
    N = wm.shape[1]
    tm = 512
    while tm > 0 and M % tm:
        tm -= 16
    if tm == 0:
        tm = M
    tn = 256 if N % 256 == 0 else N

    args = [x, wm, s, sh]
    in_specs = [
        pl.BlockSpec((tm, K), lambda i, j: (i, 0)),
        pl.BlockSpec((K, tn), lambda i, j: (0, j)),
        pl.BlockSpec((1, tn), lambda i, j: (0, j)),
        pl.BlockSpec((1, tn), lambda i, j: (0, j)),
    ]
    if residual is not None:
        args.append(residual)
        in_specs.append(pl.BlockSpec((tm, tn), lambda i, j: (i, j)))
        body = functools.partial(_gemm_res_kernel, relu=relu, mp=mp, tm=tm)
    else:
        body = functools.partial(_gemm_kernel, relu=relu, mp=mp, tm=tm)

    return pl.pallas_call(
        body,
        out_shape=jax.ShapeDtypeStruct((M, N), _BF),
        grid=(M // tm, N // tn),
        in_specs=in_specs,
        out_specs=pl.BlockSpec((tm, tn), lambda i, j: (i, j)),
        compiler_params=pltpu.CompilerParams(
            dimension_semantics=("arbitrary", "arbitrary"),
            vmem_limit_bytes=60 * 1024 * 1024),
    )(*args)


_S0 = 116 * 120
_RW0 = 120
_M0 = 112 * _RW0


def _stem_kernel(x_ref, w_ref, s_ref, h_ref, o_ref, ys, rs):
    f = x_ref[...]
    n3 = _S0 - 3
    g = jnp.concatenate(
        [f[0:n3], f[1:1 + n3], f[2:2 + n3], f[3:3 + n3]], axis=1)
    h4 = jnp.concatenate(
        [g[0:_M0], g[_RW0:_RW0 + _M0],
         g[2 * _RW0:2 * _RW0 + _M0], g[3 * _RW0:3 * _RW0 + _M0]],
        axis=1)
    acc = jnp.dot(h4, w_ref[...], preferred_element_type=_F32)
    y = jnp.maximum(acc * s_ref[...] + h_ref[...], 0.0)

    y3 = y.reshape(112, _RW0, 64)
    ys[...] = jnp.concatenate(
        [y3, jnp.zeros((112, _RW0, 64), _F32)], axis=2)
    a_ = ys[pl.ds(0, 56, 2)]
    b_ = ys[pl.ds(1, 56, 2)]
    c_ = jnp.concatenate(
        [jnp.zeros((1, _RW0, 128), _F32), b_[0:55]], axis=0)
    rs[...] = jnp.maximum(jnp.maximum(a_, b_), c_)
    e_ = rs[:, pl.ds(0, 56, 2), :]
    o_ = rs[:, pl.ds(1, 56, 2), :]
    p_ = jnp.concatenate(
        [jnp.zeros((56, 1, 128), _F32), o_[:, 0:55, :]], axis=1)
    pooled = jnp.maximum(jnp.maximum(e_, o_), p_).astype(_BF)
    row = jnp.concatenate(
        [jnp.zeros((56, 1, 128), _BF), pooled, jnp.zeros((56, 7, 128), _BF)],
        axis=1)
    zz = jnp.concatenate(
        [jnp.zeros((1, 64, 128), _BF), row, jnp.zeros((1, 64, 128), _BF)],
        axis=0)
    o_ref[...] = zz.reshape(3712, 128)


def _stem(x, cb, N):
    xn = jnp.transpose(x, (0, 2, 3, 1))
    xp = jnp.pad(xn, ((0, 0), (3, 3), (3, 3), (0, 0)))
    x2 = xp.reshape(N, 115, 2, 115, 2, 3).transpose(0, 1, 3, 2, 4, 5)
    x2 = x2.reshape(N, 115, 115, 12)
    x2 = jnp.pad(x2, ((0, 0), (0, 1), (0, 5), (0, 4))).astype(_BF)
    x2 = x2.reshape(N * _S0, 16)

    wt = cb["w"]
    wp = jnp.pad(wt, ((0, 0), (0, 0), (0, 1), (0, 1)))
    wp = wp.reshape(64, 3, 4, 2, 4, 2).transpose(2, 4, 3, 5, 1, 0)
    wp = wp.reshape(4, 4, 12, 64)
    wp = jnp.pad(wp, ((0, 0), (0, 0), (0, 4), (0, 0)))
    wst = wp.reshape(256, 64).astype(_BF)
    sst, hst = _bn_scale_shift(cb, 64)

    return pl.pallas_call(
        _stem_kernel,
        out_shape=jax.ShapeDtypeStruct((N * 3712, 128), _BF),
        grid=(N,),
        in_specs=[
            pl.BlockSpec((_S0, 16), lambda i: (i, 0)),
            pl.BlockSpec((256, 64), lambda i: (0, 0)),
            pl.BlockSpec((1, 64), lambda i: (0, 0)),
            pl.BlockSpec((1, 64), lambda i: (0, 0)),
        ],
        out_specs=pl.BlockSpec((3712, 128), lambda i: (i, 0)),
        scratch_shapes=[pltpu.VMEM((112, _RW0, 128), _F32),
                        pltpu.VMEM((56, _RW0, 128), _F32)],
        compiler_params=pltpu.CompilerParams(
            dimension_semantics=("arbitrary",),
            vmem_limit_bytes=60 * 1024 * 1024),
    )(x2, wst, sst, hst)


def _trans_kernel(x_ref, w1r, s1r, h1r, w2r, s2r, h2r, w3r, s3r, h3r,
                  wdr, sdr, hdr, o_ref, y1s3, xf, *, B, SA, rowsA, RWA,
                  rowsB, RWB, HA, WA, HB, WB, C1, Cin, Cout):
    mskA = _interior(0, B * SA, SA, RWA, HA, WA)
    xv = x_ref[...]
    x = xv.reshape(B * SA, Cin)

    a1 = jnp.dot(x, w1r[...], preferred_element_type=_F32)
    y1 = jnp.maximum(a1 * s1r[...] + h1r[...], 0.0) * mskA
    y1s3[...] = y1.reshape(B * rowsA, RWA, C1 // 128, 128)

    rA2 = B * rowsA // 2
    rI2 = rowsA // 2
    cA2 = RWA // 2
    pp = {}
    for pr in (0, 1):
        for pc in (0, 1):
            v = y1s3[pl.ds(pr, rA2, 2), pl.ds(pc, cA2, 2), :, :]
            pp[(pr, pc)] = v.reshape(rA2, cA2, C1).astype(_BF)

    M2 = B * HB * WB
    acc = jnp.zeros((M2, C1), _F32)
    for a in range(3):
        for b in range(3):
            plane = pp[(a % 2, b % 2)]
            dr, dc = a // 2, b // 2
            parts = []
            for im in range(B):
                r0 = im * rI2 + dr
                t = plane[r0:r0 + HB, dc:dc + WB, :]
                parts.append(t.reshape(HB * WB, C1))
            tap = jnp.concatenate(parts, axis=0) if B > 1 else parts[0]
            k = a * 3 + b
            acc = acc + jnp.dot(tap, w2r[k * C1:(k + 1) * C1, :],
                                preferred_element_type=_F32)
    y2 = jnp.maximum(acc * s2r[...] + h2r[...], 0.0).astype(_BF)

    xf[...] = xv.astype(_F32).reshape(B * rowsA, RWA, Cin // 128, 128)
    px = (xf[pl.ds(1, rA2, 2), pl.ds(1, cA2, 2), :, :]
          .reshape(rA2, cA2, Cin).astype(_BF))
    parts = []
    for im in range(B):
        t = px[im * rI2:im * rI2 + HB, 0:WB, :]
        parts.append(t.reshape(HB * WB, Cin))
    sub = jnp.concatenate(parts, axis=0) if B > 1 else parts[0]
    ident = (jnp.dot(sub, wdr[...], preferred_element_type=_F32)
             * sdr[...] + hdr[...])

    a3 = jnp.dot(y2, w3r[...], preferred_element_type=_F32) * s3r[...] + h3r[...]
    out = jnp.maximum(a3 + ident, 0.0).astype(_BF)

    imgs = []
    ztop = jnp.zeros((1, RWB, Cout), _BF)
    zbot = jnp.zeros((rowsB - HB - 1, RWB, Cout), _BF)
    zl = jnp.zeros((HB, 1, Cout), _BF)
    zr = jnp.zeros((HB, RWB - WB - 1, Cout), _BF)
    for im in range(B):
        blockv = out[im * HB * WB:(im + 1) * HB * WB].reshape(HB, WB, Cout)
        blockv = jnp.concatenate([zl, blockv, zr], axis=1)
        imgs.append(jnp.concatenate([ztop, blockv, zbot], axis=0))
    zz = jnp.concatenate(imgs, axis=0) if B > 1 else imgs[0]
    o_ref[...] = zz.reshape(B * rowsB * RWB, Cout)


def _transition(x, bp, HA, WA, cin_p, c1_p, cout_p, N):
    rowsA, RWA, B = _PLAN[HA]
    SA = rowsA * RWA
    HB, WB = HA // 2, WA // 2
    rowsB, RWB, _ = _PLAN[HB]
    SB = rowsB * RWB

    w1 = _w_1x1(bp["conv1"]["w"], cin_p, c1_p)
    s1, h1 = _bn_scale_shift(bp["conv1"], c1_p)
    w2 = _w_3x3(bp["conv2"]["w"], c1_p, c1_p)
    s2, h2 = _bn_scale_shift(bp["conv2"], c1_p)
    w3 = _w_1x1(bp["conv3"]["w"], c1_p, cout_p)
    s3, h3 = _bn_scale_shift(bp["conv3"], cout_p)
    wd = _w_1x1(bp["ds"]["w"], cin_p, cout_p)
    sd, hd = _bn_scale_shift(bp["ds"], cout_p)

    return pl.pallas_call(
        functools.partial(_trans_kernel, B=B, SA=SA, rowsA=rowsA, RWA=RWA,
                          rowsB=rowsB, RWB=RWB, HA=HA, WA=WA, HB=HB, WB=WB,
                          C1=c1_p, Cin=cin_p, Cout=cout_p),
        out_shape=jax.ShapeDtypeStruct((N * SB, cout_p), _BF),
        grid=(N // B,),
        in_specs=[
            pl.BlockSpec((B * rowsA, RWA, cin_p), lambda i: (i, 0, 0)),
            pl.BlockSpec((cin_p, c1_p), lambda i: (0, 0)),
            pl.BlockSpec((1, c1_p), lambda i: (0, 0)),
            pl.BlockSpec((1, c1_p), lambda i: (0, 0)),
            pl.BlockSpec((9 * c1_p, c1_p), lambda i: (0, 0)),
            pl.BlockSpec((1, c1_p), lambda i: (0, 0)),
            pl.BlockSpec((1, c1_p), lambda i: (0, 0)),
            pl.BlockSpec((c1_p, cout_p), lambda i: (0, 0)),
            pl.BlockSpec((1, cout_p), lambda i: (0, 0)),
            pl.BlockSpec((1, cout_p), lambda i: (0, 0)),
            pl.BlockSpec((cin_p, cout_p), lambda i: (0, 0)),
            pl.BlockSpec((1, cout_p), lambda i: (0, 0)),
            pl.BlockSpec((1, cout_p), lambda i: (0, 0)),
        ],
        out_specs=pl.BlockSpec((B * SB, cout_p), lambda i: (i, 0)),
        scratch_shapes=[
            pltpu.VMEM((B * rowsA, RWA, c1_p // 128, 128), _F32),
            pltpu.VMEM((B * rowsA, RWA, cin_p // 128, 128), _F32)],
        compiler_params=pltpu.CompilerParams(
            dimension_semantics=("arbitrary",),
            vmem_limit_bytes=60 * 1024 * 1024),
    )(x.reshape(N * rowsA, RWA, cin_p), w1, s1, h1, w2, s2, h2, w3, s3, h3,
      wd, sd, hd)


def _fc_kernel(a_ref, x_ref, w_ref, b_ref, o_ref, *, inv_s):
    t = jnp.dot(a_ref[...], x_ref[...], preferred_element_type=_F32)
    xm = (t * inv_s).astype(_BF)
    o_ref[...] = jnp.dot(xm, w_ref[...], preferred_element_type=_F32) + b_ref[...]


def _avgpool_fc(x, fc_w, fc_b, S, H, W, N):
    C = x.shape[1]
    nc = fc_w.shape[0]
    ncp = _ru(nc, 128)
    amat = jnp.repeat(jnp.eye(N, dtype=_BF), S, axis=1)
    wm = jnp.pad(jnp.transpose(fc_w), ((0, 0), (0, ncp - nc))).astype(_BF)
    bv = jnp.pad(fc_b, (0, ncp - nc)).reshape(1, ncp).astype(_F32)
    out = pl.pallas_call(
        functools.partial(_fc_kernel, inv_s=1.0 / float(H * W)),
        out_shape=jax.ShapeDtypeStruct((N, ncp), _F32),
        grid=(1,),
        in_specs=[
            pl.BlockSpec((N, N * S), lambda i: (0, 0)),
            pl.BlockSpec((N * S, C), lambda i: (0, 0)),
            pl.BlockSpec((C, ncp), lambda i: (0, 0)),
            pl.BlockSpec((1, ncp), lambda i: (0, 0)),
        ],
        out_specs=pl.BlockSpec((N, ncp), lambda i: (0, 0)),
        compiler_params=pltpu.CompilerParams(
            dimension_semantics=("arbitrary",),
            vmem_limit_bytes=60 * 1024 * 1024),
    )(amat, x, wm, bv)
    return out[:, :nc]


_SUF = ("w", "b", "gamma", "beta", "mean", "var")


def kernel(*args):
    x = args[0]
    stem = dict(zip(_SUF, args[1:7]))
    idx = 7
    layers = []
    for nblocks in (3, 4, 6, 3):
        blocks = []
        for b in range(nblocks):
            bp = {}
            for cname in ("conv1", "conv2", "conv3"):
                bp[cname] = dict(zip(_SUF, args[idx:idx + 6]))
                idx += 6
            if b == 0:
                bp["ds"] = dict(zip(_SUF, args[idx:idx + 6]))
                idx += 6
            blocks.append(bp)
        layers.append(blocks)
    fc_w, fc_b = args[idx], args[idx + 1]

    N = x.shape[0]

    h = _stem(x, stem, N)

    cfg = [(56, 128, 128, 256), (28, 256, 128, 512),
           (14, 512, 256, 1024), (7, 1024, 512, 2048)]
    for li, (H, cin_p, c1_p, cout_p) in enumerate(cfg):
        blocks = layers[li]
        if li == 0:
            h = _bottleneck(h, blocks[0]["conv1"], blocks[0]["conv2"],
                            blocks[0]["conv3"], blocks[0]["ds"],
                            H, H, cin_p, c1_p, cout_p, N)
        else:
            h = _transition(h, blocks[0], H * 2, H * 2, cin_p, c1_p, cout_p, N)
        for bp in blocks[1:]:
            h = _bottleneck(h, bp["conv1"], bp["conv2"], bp["conv3"], None,
                            H, H, cout_p, c1_p, cout_p, N)

    rows4, RW4, _ = _PLAN[7]
    return _avgpool_fc(h, fc_w, fc_b, rows4 * RW4, 7, 7, N)


# speedup vs baseline: 8.2724x; 1.0000x over previous
import functools

import jax
import jax.numpy as jnp
from jax.experimental import pallas as pl
from jax.experimental.pallas import tpu as pltpu

_EPS = 1e-5
_BF = jnp.bfloat16
_F32 = jnp.float32

_PLAN = {56: (58, 64, 1), 28: (30, 32, 4), 14: (16, 16, 8), 7: (10, 9, 8)}


def _ru(x, m):
    return ((x + m - 1) // m) * m


def _bn_scale_shift(cb, cout_p):
    s = cb["gamma"] * jax.lax.rsqrt(cb["var"] + _EPS)
    sh = cb["b"] * s + cb["beta"] - cb["mean"] * s
    cout = s.shape[0]
    if cout_p != cout:
        s = jnp.pad(s, (0, cout_p - cout))
        sh = jnp.pad(sh, (0, cout_p - cout))
    return (s.reshape(1, cout_p).astype(_F32),
            sh.reshape(1, cout_p).astype(_F32))


def _w_1x1(w, cin_p, cout_p):
    cout, cin = w.shape[0], w.shape[1]
    wm = jnp.transpose(w[:, :, 0, 0])
    wm = jnp.pad(wm, ((0, cin_p - cin), (0, cout_p - cout)))
    return wm.astype(_BF)


def _w_3x3(w, cin_p, cout_p):
    cout, cin = w.shape[0], w.shape[1]
    wt = jnp.transpose(w, (2, 3, 1, 0))
    wt = jnp.pad(wt, ((0, 0), (0, 0), (0, cin_p - cin), (0, cout_p - cout)))
    return wt.reshape(9 * cin_p, cout_p).astype(_BF)


def _interior(mstart, bs, S, RW, H, W):
    m = jax.lax.broadcasted_iota(jnp.int32, (bs, 1), 0) + mstart
    p = m % S
    r = p // RW
    c = p % RW
    ok = (r >= 1) & (r <= H) & (c >= 1) & (c <= W)
    return ok.astype(_F32)


def _bneck_kernel(*refs, has_ds, S, RW, H, W, C1, PAD, BS):
    if has_ds:
        (x_ref, w1r, s1r, h1r, w2r, s2r, h2r, w3r, s3r, h3r,
         wdr, sdr, hdr, o_ref, y1s) = refs
    else:
        (x_ref, w1r, s1r, h1r, w2r, s2r, h2r, w3r, s3r, h3r,
         o_ref, y1s) = refs

    msk = _interior(0, BS, S, RW, H, W)
    x = x_ref[...]

    a1 = jnp.dot(x, w1r[...], preferred_element_type=_F32)
    y1 = jnp.maximum(a1 * s1r[...] + h1r[...], 0.0) * msk
    y1s[0:PAD, :] = jnp.zeros((PAD, C1), _BF)
    y1s[PAD + BS:, :] = jnp.zeros((PAD, C1), _BF)
    y1s[PAD:PAD + BS, :] = y1.astype(_BF)

    acc = jnp.zeros((BS, C1), _F32)
    for a in range(3):
        for b in range(3):
            off = PAD + (a - 1) * RW + (b - 1)
            k = a * 3 + b
            acc = acc + jnp.dot(y1s[off:off + BS, :],
                                w2r[k * C1:(k + 1) * C1, :],
                                preferred_element_type=_F32)
    y2 = (jnp.maximum(acc * s2r[...] + h2r[...], 0.0) * msk).astype(_BF)

    a3 = jnp.dot(y2, w3r[...], preferred_element_type=_F32) * s3r[...] + h3r[...]
    if has_ds:
        ident = (jnp.dot(x, wdr[...], preferred_element_type=_F32)
                 * sdr[...] + hdr[...])
    else:
        ident = x.astype(_F32)
    o_ref[...] = (jnp.maximum(a3 + ident, 0.0) * msk).astype(_BF)


def _bottleneck(x, p1, p2, p3, pds, H, W, cin_p, c1_p, cout_p, N):
    rows, RW, B = _PLAN[H]
    S = rows * RW
    BS = B * S
    PAD = _ru(RW + 1, 8)

    w1 = _w_1x1(p1["w"], cin_p, c1_p)
    s1, h1 = _bn_scale_shift(p1, c1_p)
    w2 = _w_3x3(p2["w"], c1_p, c1_p)
    s2, h2 = _bn_scale_shift(p2, c1_p)
    w3 = _w_1x1(p3["w"], c1_p, cout_p)
    s3, h3 = _bn_scale_shift(p3, cout_p)

    args = [x, w1, s1, h1, w2, s2, h2, w3, s3, h3]
    in_specs = [
        pl.BlockSpec((BS, cin_p), lambda i: (i, 0)),
        pl.BlockSpec((cin_p, c1_p), lambda i: (0, 0)),
        pl.BlockSpec((1, c1_p), lambda i: (0, 0)),
        pl.BlockSpec((1, c1_p), lambda i: (0, 0)),
        pl.BlockSpec((9 * c1_p, c1_p), lambda i: (0, 0)),
        pl.BlockSpec((1, c1_p), lambda i: (0, 0)),
        pl.BlockSpec((1, c1_p), lambda i: (0, 0)),
        pl.BlockSpec((c1_p, cout_p), lambda i: (0, 0)),
        pl.BlockSpec((1, cout_p), lambda i: (0, 0)),
        pl.BlockSpec((1, cout_p), lambda i: (0, 0)),
    ]
    if pds is not None:
        wds = _w_1x1(pds["w"], cin_p, cout_p)
        sds, hds = _bn_scale_shift(pds, cout_p)
        args += [wds, sds, hds]
        in_specs += [
            pl.BlockSpec((cin_p, cout_p), lambda i: (0, 0)),
            pl.BlockSpec((1, cout_p), lambda i: (0, 0)),
            pl.BlockSpec((1, cout_p), lambda i: (0, 0)),
        ]

    return pl.pallas_call(
        functools.partial(_bneck_kernel, has_ds=pds is not None, S=S, RW=RW,
                          H=H, W=W, C1=c1_p, PAD=PAD, BS=BS),
        out_shape=jax.ShapeDtypeStruct((N * S, cout_p), _BF),
        grid=(N // B,),
        in_specs=in_specs,
        out_specs=pl.BlockSpec((BS, cout_p), lambda i: (i, 0)),
        scratch_shapes=[pltpu.VMEM((PAD + BS + PAD, c1_p), _BF)],
        compiler_params=pltpu.CompilerParams(
            dimension_semantics=("arbitrary",),
            vmem_limit_bytes=60 * 1024 * 1024),
    )(*args)


def _gemm_kernel(x_ref, w_ref, s_ref, h_ref, o_ref, *, relu, mp, tm):
    y = jnp.dot(x_ref[...], w_ref[...], preferred_element_type=_F32)
    y = y * s_ref[...] + h_ref[...]
    if relu:
        y = jnp.maximum(y, 0.0)
    if mp is not None:
        S, RW, H, W = mp
        y = y * _interior(pl.program_id(0) * tm, tm, S, RW, H, W)
    o_ref[...] = y.astype(o_ref.dtype)


def _gemm_res_kernel(x_ref, w_ref, s_ref, h_ref, r_ref, o_ref, *, relu, mp, tm):
    y = jnp.dot(x_ref[...], w_ref[...], preferred_element_type=_F32)
    y = y * s_ref[...] + h_ref[...] + r_ref[...].astype(_F32)
    if relu:
        y = jnp.maximum(y, 0.0)
    if mp is not None:
        S, RW, H, W = mp
        y = y * _interior(pl.program_id(0) * tm, tm, S, RW, H, W)
    o_ref[...] = y.astype(o_ref.dtype)


def _gemm(x, wm, s, sh, residual=None, relu=True, mp=None):
    M, K = x.shape
    N = wm.shape[1]
    tm = 512
    while tm > 0 and M % tm:
        tm -= 16
    if tm == 0:
        tm = M
    tn = 256 if N % 256 == 0 else N

    args = [x, wm, s, sh]
    in_specs = [
        pl.BlockSpec((tm, K), lambda i, j: (i, 0)),
        pl.BlockSpec((K, tn), lambda i, j: (0, j)),
        pl.BlockSpec((1, tn), lambda i, j: (0, j)),
        pl.BlockSpec((1, tn), lambda i, j: (0, j)),
    ]
    if residual is not None:
        args.append(residual)
        in_specs.append(pl.BlockSpec((tm, tn), lambda i, j: (i, j)))
        body = functools.partial(_gemm_res_kernel, relu=relu, mp=mp, tm=tm)
    else:
        body = functools.partial(_gemm_kernel, relu=relu, mp=mp, tm=tm)

    return pl.pallas_call(
        body,
        out_shape=jax.ShapeDtypeStruct((M, N), _BF),
        grid=(M // tm, N // tn),
        in_specs=in_specs,
        out_specs=pl.BlockSpec((tm, tn), lambda i, j: (i, j)),
        compiler_params=pltpu.CompilerParams(
            dimension_semantics=("arbitrary", "arbitrary"),
            vmem_limit_bytes=60 * 1024 * 1024),
    )(*args)


_S0 = 116 * 120
_RW0 = 120
_M0 = 112 * _RW0


def _stem_kernel(x_ref, w_ref, s_ref, h_ref, o_ref, ys, rs):
    f = x_ref[...]
    n3 = _S0 - 3
    g = jnp.concatenate(
        [f[0:n3], f[1:1 + n3], f[2:2 + n3], f[3:3 + n3]], axis=1)
    h4 = jnp.concatenate(
        [g[0:_M0], g[_RW0:_RW0 + _M0],
         g[2 * _RW0:2 * _RW0 + _M0], g[3 * _RW0:3 * _RW0 + _M0]],
        axis=1)
    acc = jnp.dot(h4, w_ref[...], preferred_element_type=_F32)
    y = jnp.maximum(acc * s_ref[...] + h_ref[...], 0.0)

    y3 = y.reshape(112, _RW0, 64)
    ys[...] = jnp.concatenate(
        [y3, jnp.zeros((112, _RW0, 64), _F32)], axis=2)
    a_ = ys[pl.ds(0, 56, 2)]
    b_ = ys[pl.ds(1, 56, 2)]
    c_ = jnp.concatenate(
        [jnp.zeros((1, _RW0, 128), _F32), b_[0:55]], axis=0)
    rs[...] = jnp.maximum(jnp.maximum(a_, b_), c_)
    e_ = rs[:, pl.ds(0, 56, 2), :]
    o_ = rs[:, pl.ds(1, 56, 2), :]
    p_ = jnp.concatenate(
        [jnp.zeros((56, 1, 128), _F32), o_[:, 0:55, :]], axis=1)
    pooled = jnp.maximum(jnp.maximum(e_, o_), p_).astype(_BF)
    row = jnp.concatenate(
        [jnp.zeros((56, 1, 128), _BF), pooled, jnp.zeros((56, 7, 128), _BF)],
        axis=1)
    zz = jnp.concatenate(
        [jnp.zeros((1, 64, 128), _BF), row, jnp.zeros((1, 64, 128), _BF)],
        axis=0)
    o_ref[...] = zz.reshape(3712, 128)


def _stem(x, cb, N):
    xn = jnp.transpose(x.astype(_BF), (0, 2, 3, 1))
    xp = jnp.pad(xn, ((0, 0), (3, 3), (3, 3), (0, 0)))
    x2 = xp.reshape(N, 115, 2, 115, 2, 3).transpose(0, 1, 3, 2, 4, 5)
    x2 = x2.reshape(N, 115, 115, 12)
    x2 = jnp.pad(x2, ((0, 0), (0, 1), (0, 5), (0, 4)))
    x2 = x2.reshape(N * _S0, 16)

    wt = cb["w"]
    wp = jnp.pad(wt, ((0, 0), (0, 0), (0, 1), (0, 1)))
    wp = wp.reshape(64, 3, 4, 2, 4, 2).transpose(2, 4, 3, 5, 1, 0)
    wp = wp.reshape(4, 4, 12, 64)
    wp = jnp.pad(wp, ((0, 0), (0, 0), (0, 4), (0, 0)))
    wst = wp.reshape(256, 64).astype(_BF)
    sst, hst = _bn_scale_shift(cb, 64)

    return pl.pallas_call(
        _stem_kernel,
        out_shape=jax.ShapeDtypeStruct((N * 3712, 128), _BF),
        grid=(N,),
        in_specs=[
            pl.BlockSpec((_S0, 16), lambda i: (i, 0)),
            pl.BlockSpec((256, 64), lambda i: (0, 0)),
            pl.BlockSpec((1, 64), lambda i: (0, 0)),
            pl.BlockSpec((1, 64), lambda i: (0, 0)),
        ],
        out_specs=pl.BlockSpec((3712, 128), lambda i: (i, 0)),
        scratch_shapes=[pltpu.VMEM((112, _RW0, 128), _F32),
                        pltpu.VMEM((56, _RW0, 128), _F32)],
        compiler_params=pltpu.CompilerParams(
            dimension_semantics=("arbitrary",),
            vmem_limit_bytes=60 * 1024 * 1024),
    )(x2, wst, sst, hst)


def _trans_kernel(x_ref, w1r, s1r, h1r, w2r, s2r, h2r, w3r, s3r, h3r,
                  wdr, sdr, hdr, o_ref, y1s3, xf, *, B, SA, rowsA, RWA,
                  rowsB, RWB, HA, WA, HB, WB, C1, Cin, Cout):
    mskA = _interior(0, B * SA, SA, RWA, HA, WA)
    xv = x_ref[...]
    x = xv.reshape(B * SA, Cin)

    a1 = jnp.dot(x, w1r[...], preferred_element_type=_F32)
    y1 = jnp.maximum(a1 * s1r[...] + h1r[...], 0.0) * mskA
    y1s3[...] = y1.reshape(B * rowsA, RWA, C1 // 128, 128)

    rA2 = B * rowsA // 2
    rI2 = rowsA // 2
    cA2 = RWA // 2
    pp = {}
    for pr in (0, 1):
        for pc in (0, 1):
            v = y1s3[pl.ds(pr, rA2, 2), pl.ds(pc, cA2, 2), :, :]
            pp[(pr, pc)] = v.reshape(rA2, cA2, C1).astype(_BF)

    M2 = B * HB * WB
    acc = jnp.zeros((M2, C1), _F32)
    for a in range(3):
        for b in range(3):
            plane = pp[(a % 2, b % 2)]
            dr, dc = a // 2, b // 2
            parts = []
            for im in range(B):
                r0 = im * rI2 + dr
                t = plane[r0:r0 + HB, dc:dc + WB, :]
                parts.append(t.reshape(HB * WB, C1))
            tap = jnp.concatenate(parts, axis=0) if B > 1 else parts[0]
            k = a * 3 + b
            acc = acc + jnp.dot(tap, w2r[k * C1:(k + 1) * C1, :],
                                preferred_element_type=_F32)
    y2 = jnp.maximum(acc * s2r[...] + h2r[...], 0.0).astype(_BF)

    xf[...] = xv.astype(_F32).reshape(B * rowsA, RWA, Cin // 128, 128)
    px = (xf[pl.ds(1, rA2, 2), pl.ds(1, cA2, 2), :, :]
          .reshape(rA2, cA2, Cin).astype(_BF))
    parts = []
    for im in range(B):
        t = px[im * rI2:im * rI2 + HB, 0:WB, :]
        parts.append(t.reshape(HB * WB, Cin))
    sub = jnp.concatenate(parts, axis=0) if B > 1 else parts[0]
    ident = (jnp.dot(sub, wdr[...], preferred_element_type=_F32)
             * sdr[...] + hdr[...])

    a3 = jnp.dot(y2, w3r[...], preferred_element_type=_F32) * s3r[...] + h3r[...]
    out = jnp.maximum(a3 + ident, 0.0).astype(_BF)

    imgs = []
    ztop = jnp.zeros((1, RWB, Cout), _BF)
    zbot = jnp.zeros((rowsB - HB - 1, RWB, Cout), _BF)
    zl = jnp.zeros((HB, 1, Cout), _BF)
    zr = jnp.zeros((HB, RWB - WB - 1, Cout), _BF)
    for im in range(B):
        blockv = out[im * HB * WB:(im + 1) * HB * WB].reshape(HB, WB, Cout)
        blockv = jnp.concatenate([zl, blockv, zr], axis=1)
        imgs.append(jnp.concatenate([ztop, blockv, zbot], axis=0))
    zz = jnp.concatenate(imgs, axis=0) if B > 1 else imgs[0]
    o_ref[...] = zz.reshape(B * rowsB * RWB, Cout)


def _transition(x, bp, HA, WA, cin_p, c1_p, cout_p, N):
    rowsA, RWA, B = _PLAN[HA]
    SA = rowsA * RWA
    HB, WB = HA // 2, WA // 2
    rowsB, RWB, _ = _PLAN[HB]
    SB = rowsB * RWB

    w1 = _w_1x1(bp["conv1"]["w"], cin_p, c1_p)
    s1, h1 = _bn_scale_shift(bp["conv1"], c1_p)
    w2 = _w_3x3(bp["conv2"]["w"], c1_p, c1_p)
    s2, h2 = _bn_scale_shift(bp["conv2"], c1_p)
    w3 = _w_1x1(bp["conv3"]["w"], c1_p, cout_p)
    s3, h3 = _bn_scale_shift(bp["conv3"], cout_p)
    wd = _w_1x1(bp["ds"]["w"], cin_p, cout_p)
    sd, hd = _bn_scale_shift(bp["ds"], cout_p)

    return pl.pallas_call(
        functools.partial(_trans_kernel, B=B, SA=SA, rowsA=rowsA, RWA=RWA,
                          rowsB=rowsB, RWB=RWB, HA=HA, WA=WA, HB=HB, WB=WB,
                          C1=c1_p, Cin=cin_p, Cout=cout_p),
        out_shape=jax.ShapeDtypeStruct((N * SB, cout_p), _BF),
        grid=(N // B,),
        in_specs=[
            pl.BlockSpec((B * rowsA, RWA, cin_p), lambda i: (i, 0, 0)),
            pl.BlockSpec((cin_p, c1_p), lambda i: (0, 0)),
            pl.BlockSpec((1, c1_p), lambda i: (0, 0)),
            pl.BlockSpec((1, c1_p), lambda i: (0, 0)),
            pl.BlockSpec((9 * c1_p, c1_p), lambda i: (0, 0)),
            pl.BlockSpec((1, c1_p), lambda i: (0, 0)),
            pl.BlockSpec((1, c1_p), lambda i: (0, 0)),
            pl.BlockSpec((c1_p, cout_p), lambda i: (0, 0)),
            pl.BlockSpec((1, cout_p), lambda i: (0, 0)),
            pl.BlockSpec((1, cout_p), lambda i: (0, 0)),
            pl.BlockSpec((cin_p, cout_p), lambda i: (0, 0)),
            pl.BlockSpec((1, cout_p), lambda i: (0, 0)),
            pl.BlockSpec((1, cout_p), lambda i: (0, 0)),
        ],
        out_specs=pl.BlockSpec((B * SB, cout_p), lambda i: (i, 0)),
        scratch_shapes=[
            pltpu.VMEM((B * rowsA, RWA, c1_p // 128, 128), _F32),
            pltpu.VMEM((B * rowsA, RWA, cin_p // 128, 128), _F32)],
        compiler_params=pltpu.CompilerParams(
            dimension_semantics=("arbitrary",),
            vmem_limit_bytes=60 * 1024 * 1024),
    )(x.reshape(N * rowsA, RWA, cin_p), w1, s1, h1, w2, s2, h2, w3, s3, h3,
      wd, sd, hd)


def _fc_kernel(a_ref, x_ref, w_ref, b_ref, o_ref, *, inv_s):
    t = jnp.dot(a_ref[...], x_ref[...], preferred_element_type=_F32)
    xm = (t * inv_s).astype(_BF)
    o_ref[...] = jnp.dot(xm, w_ref[...], preferred_element_type=_F32) + b_ref[...]


def _avgpool_fc(x, fc_w, fc_b, S, H, W, N):
    C = x.shape[1]
    nc = fc_w.shape[0]
    ncp = _ru(nc, 128)
    amat = jnp.repeat(jnp.eye(N, dtype=_BF), S, axis=1)
    wm = jnp.pad(jnp.transpose(fc_w), ((0, 0), (0, ncp - nc))).astype(_BF)
    bv = jnp.pad(fc_b, (0, ncp - nc)).reshape(1, ncp).astype(_F32)
    out = pl.pallas_call(
        functools.partial(_fc_kernel, inv_s=1.0 / float(H * W)),
        out_shape=jax.ShapeDtypeStruct((N, ncp), _F32),
        grid=(1,),
        in_specs=[
            pl.BlockSpec((N, N * S), lambda i: (0, 0)),
            pl.BlockSpec((N * S, C), lambda i: (0, 0)),
            pl.BlockSpec((C, ncp), lambda i: (0, 0)),
            pl.BlockSpec((1, ncp), lambda i: (0, 0)),
        ],
        out_specs=pl.BlockSpec((N, ncp), lambda i: (0, 0)),
        compiler_params=pltpu.CompilerParams(
            dimension_semantics=("arbitrary",),
            vmem_limit_bytes=60 * 1024 * 1024),
    )(amat, x, wm, bv)
    return out[:, :nc]


_SUF = ("w", "b", "gamma", "beta", "mean", "var")


def kernel(*args):
    x = args[0]
    stem = dict(zip(_SUF, args[1:7]))
    idx = 7
    layers = []
    for nblocks in (3, 4, 6, 3):
        blocks = []
        for b in range(nblocks):
            bp = {}
            for cname in ("conv1", "conv2", "conv3"):
                bp[cname] = dict(zip(_SUF, args[idx:idx + 6]))
                idx += 6
            if b == 0:
                bp["ds"] = dict(zip(_SUF, args[idx:idx + 6]))
                idx += 6
            blocks.append(bp)
        layers.append(blocks)
    fc_w, fc_b = args[idx], args[idx + 1]

    N = x.shape[0]

    h = _stem(x, stem, N)

    cfg = [(56, 128, 128, 256), (28, 256, 128, 512),
           (14, 512, 256, 1024), (7, 1024, 512, 2048)]
    for li, (H, cin_p, c1_p, cout_p) in enumerate(cfg):
        blocks = layers[li]
        if li == 0:
            h = _bottleneck(h, blocks[0]["conv1"], blocks[0]["conv2"],
                            blocks[0]["conv3"], blocks[0]["ds"],
                            H, H, cin_p, c1_p, cout_p, N)
        else:
            h = _transition(h, blocks[0], H * 2, H * 2, cin_p, c1_p, cout_p, N)
        for bp in blocks[1:]:
            h = _bottleneck(h, bp["conv1"], bp["conv2"], bp["conv3"], None,
                            H, H, cout_p, c1_p, cout_p, N)

    rows4, RW4, _ = _PLAN[7]
    return _avgpool_fc(h, fc_w, fc_b, rows4 * RW4, 7, 7, N)


# speedup vs baseline: 8.3835x; 1.0134x over previous
import functools

import jax
import jax.numpy as jnp
from jax.experimental import pallas as pl
from jax.experimental.pallas import tpu as pltpu

_EPS = 1e-5
_BF = jnp.bfloat16
_F32 = jnp.float32

_PLAN = {56: (58, 64, 2), 28: (30, 32, 4), 14: (16, 16, 8), 7: (10, 9, 8)}


def _ru(x, m):
    return ((x + m - 1) // m) * m


def _bn_scale_shift(cb, cout_p):
    s = cb["gamma"] * jax.lax.rsqrt(cb["var"] + _EPS)
    sh = cb["b"] * s + cb["beta"] - cb["mean"] * s
    cout = s.shape[0]
    if cout_p != cout:
        s = jnp.pad(s, (0, cout_p - cout))
        sh = jnp.pad(sh, (0, cout_p - cout))
    return (s.reshape(1, cout_p).astype(_F32),
            sh.reshape(1, cout_p).astype(_F32))


def _w_1x1(w, cin_p, cout_p):
    cout, cin = w.shape[0], w.shape[1]
    wm = jnp.transpose(w[:, :, 0, 0])
    wm = jnp.pad(wm, ((0, cin_p - cin), (0, cout_p - cout)))
    return wm.astype(_BF)


def _w_3x3(w, cin_p, cout_p):
    cout, cin = w.shape[0], w.shape[1]
    wt = jnp.transpose(w, (2, 3, 1, 0))
    wt = jnp.pad(wt, ((0, 0), (0, 0), (0, cin_p - cin), (0, cout_p - cout)))
    return wt.reshape(9 * cin_p, cout_p).astype(_BF)


def _interior(mstart, bs, S, RW, H, W):
    m = jax.lax.broadcasted_iota(jnp.int32, (bs, 1), 0) + mstart
    p = m % S
    r = p // RW
    c = p % RW
    ok = (r >= 1) & (r <= H) & (c >= 1) & (c <= W)
    return ok.astype(_F32)


def _bneck_kernel(*refs, has_ds, S, RW, H, W, C1, PAD, BS):
    if has_ds:
        (x_ref, w1r, s1r, h1r, w2r, s2r, h2r, w3r, s3r, h3r,
         wdr, sdr, hdr, o_ref, y1s, yws) = refs
    else:
        (x_ref, w1r, s1r, h1r, w2r, s2r, h2r, w3r, s3r, h3r,
         o_ref, y1s, yws) = refs

    msk = _interior(0, BS, S, RW, H, W)
    x = x_ref[...]

    a1 = jnp.dot(x, w1r[...], preferred_element_type=_F32)
    y1 = jnp.maximum(a1 * s1r[...] + h1r[...], 0.0) * msk
    y1s[0:PAD, :] = jnp.zeros((PAD, C1), _BF)
    y1s[PAD + BS:, :] = jnp.zeros((PAD, C1), _BF)
    y1s[PAD:PAD + BS, :] = y1.astype(_BF)

    yw = jnp.concatenate(
        [y1s[PAD - 1:PAD - 1 + BS, :], y1s[PAD:PAD + BS, :],
         y1s[PAD + 1:PAD + 1 + BS, :]], axis=1)
    yws[0:PAD, :] = jnp.zeros((PAD, 3 * C1), _BF)
    yws[PAD + BS:, :] = jnp.zeros((PAD, 3 * C1), _BF)
    yws[PAD:PAD + BS, :] = yw
    acc = jnp.zeros((BS, C1), _F32)
    for a in range(3):
        off = PAD + (a - 1) * RW
        acc = acc + jnp.dot(yws[off:off + BS, :],
                            w2r[a * 3 * C1:(a + 1) * 3 * C1, :],
                            preferred_element_type=_F32)
    y2 = jnp.maximum(acc * s2r[...] + h2r[...], 0.0).astype(_BF)

    a3 = jnp.dot(y2, w3r[...], preferred_element_type=_F32) * s3r[...] + h3r[...]
    if has_ds:
        ident = (jnp.dot(x, wdr[...], preferred_element_type=_F32)
                 * sdr[...] + hdr[...])
    else:
        ident = x.astype(_F32)
    o_ref[...] = (jnp.maximum(a3 + ident, 0.0) * msk).astype(_BF)


def _bottleneck(x, p1, p2, p3, pds, H, W, cin_p, c1_p, cout_p, N):
    rows, RW, B = _PLAN[H]
    S = rows * RW
    BS = B * S
    PAD = _ru(RW + 1, 8)

    w1 = _w_1x1(p1["w"], cin_p, c1_p)
    s1, h1 = _bn_scale_shift(p1, c1_p)
    w2 = _w_3x3(p2["w"], c1_p, c1_p)
    s2, h2 = _bn_scale_shift(p2, c1_p)
    w3 = _w_1x1(p3["w"], c1_p, cout_p)
    s3, h3 = _bn_scale_shift(p3, cout_p)

    args = [x, w1, s1, h1, w2, s2, h2, w3, s3, h3]
    in_specs = [
        pl.BlockSpec((BS, cin_p), lambda i: (i, 0)),
        pl.BlockSpec((cin_p, c1_p), lambda i: (0, 0)),
        pl.BlockSpec((1, c1_p), lambda i: (0, 0)),
        pl.BlockSpec((1, c1_p), lambda i: (0, 0)),
        pl.BlockSpec((9 * c1_p, c1_p), lambda i: (0, 0)),
        pl.BlockSpec((1, c1_p), lambda i: (0, 0)),
        pl.BlockSpec((1, c1_p), lambda i: (0, 0)),
        pl.BlockSpec((c1_p, cout_p), lambda i: (0, 0)),
        pl.BlockSpec((1, cout_p), lambda i: (0, 0)),
        pl.BlockSpec((1, cout_p), lambda i: (0, 0)),
    ]
    if pds is not None:
        wds = _w_1x1(pds["w"], cin_p, cout_p)
        sds, hds = _bn_scale_shift(pds, cout_p)
        args += [wds, sds, hds]
        in_specs += [
            pl.BlockSpec((cin_p, cout_p), lambda i: (0, 0)),
            pl.BlockSpec((1, cout_p), lambda i: (0, 0)),
            pl.BlockSpec((1, cout_p), lambda i: (0, 0)),
        ]

    return pl.pallas_call(
        functools.partial(_bneck_kernel, has_ds=pds is not None, S=S, RW=RW,
                          H=H, W=W, C1=c1_p, PAD=PAD, BS=BS),
        out_shape=jax.ShapeDtypeStruct((N * S, cout_p), _BF),
        grid=(N // B,),
        in_specs=in_specs,
        out_specs=pl.BlockSpec((BS, cout_p), lambda i: (i, 0)),
        scratch_shapes=[pltpu.VMEM((PAD + BS + PAD, c1_p), _BF),
                        pltpu.VMEM((PAD + BS + PAD, 3 * c1_p), _BF)],
        compiler_params=pltpu.CompilerParams(
            dimension_semantics=("arbitrary",),
            vmem_limit_bytes=60 * 1024 * 1024),
    )(*args)


def _gemm_kernel(x_ref, w_ref, s_ref, h_ref, o_ref, *, relu, mp, tm):
    y = jnp.dot(x_ref[...], w_ref[...], preferred_element_type=_F32)
    y = y * s_ref[...] + h_ref[...]
    if relu:
        y = jnp.maximum(y, 0.0)
    if mp is not None:
        S, RW, H, W = mp
        y = y * _interior(pl.program_id(0) * tm, tm, S, RW, H, W)
    o_ref[...] = y.astype(o_ref.dtype)


def _gemm_res_kernel(x_ref, w_ref, s_ref, h_ref, r_ref, o_ref, *, relu, mp, tm):
    y = jnp.dot(x_ref[...], w_ref[...], preferred_element_type=_F32)
    y = y * s_ref[...] + h_ref[...] + r_ref[...].astype(_F32)
    if relu:
        y = jnp.maximum(y, 0.0)
    if mp is not None:
        S, RW, H, W = mp
        y = y * _interior(pl.program_id(0) * tm, tm, S, RW, H, W)
    o_ref[...] = y.astype(o_ref.dtype)


def _gemm(x, wm, s, sh, residual=None, relu=True, mp=None):
    M, K = x.shape
    N = wm.shape[1]
    tm = 512
    while tm > 0 and M % tm:
        tm -= 16
    if tm == 0:
        tm = M
    tn = 256 if N % 256 == 0 else N

    args = [x, wm, s, sh]
    in_specs = [
        pl.BlockSpec((tm, K), lambda i, j: (i, 0)),
        pl.BlockSpec((K, tn), lambda i, j: (0, j)),
        pl.BlockSpec((1, tn), lambda i, j: (0, j)),
        pl.BlockSpec((1, tn), lambda i, j: (0, j)),
    ]
    if residual is not None:
        args.append(residual)
        in_specs.append(pl.BlockSpec((tm, tn), lambda i, j: (i, j)))
        body = functools.partial(_gemm_res_kernel, relu=relu, mp=mp, tm=tm)
    else:
        body = functools.partial(_gemm_kernel, relu=relu, mp=mp, tm=tm)

    return pl.pallas_call(
        body,
        out_shape=jax.ShapeDtypeStruct((M, N), _BF),
        grid=(M // tm, N // tn),
        in_specs=in_specs,
        out_specs=pl.BlockSpec((tm, tn), lambda i, j: (i, j)),
        compiler_params=pltpu.CompilerParams(
            dimension_semantics=("arbitrary", "arbitrary"),
            vmem_limit_bytes=60 * 1024 * 1024),
    )(*args)


_S0 = 116 * 120
_RW0 = 120
_M0 = 112 * _RW0


def _stem_kernel(x_ref, w_ref, s_ref, h_ref, o_ref, ys, rs):
    f = x_ref[...]
    n3 = _S0 - 3
    g = jnp.concatenate(
        [f[0:n3], f[1:1 + n3], f[2:2 + n3], f[3:3 + n3]], axis=1)
    h4 = jnp.concatenate(
        [g[0:_M0], g[_RW0:_RW0 + _M0],
         g[2 * _RW0:2 * _RW0 + _M0], g[3 * _RW0:3 * _RW0 + _M0]],
        axis=1)
    acc = jnp.dot(h4, w_ref[...], preferred_element_type=_F32)
    y = jnp.maximum(acc * s_ref[...] + h_ref[...], 0.0)

    y3 = y.reshape(112, _RW0, 64)
    ys[...] = jnp.concatenate(
        [y3, jnp.zeros((112, _RW0, 64), _F32)], axis=2)
    a_ = ys[pl.ds(0, 56, 2)]
    b_ = ys[pl.ds(1, 56, 2)]
    c_ = jnp.concatenate(
        [jnp.zeros((1, _RW0, 128), _F32), b_[0:55]], axis=0)
    rs[...] = jnp.maximum(jnp.maximum(a_, b_), c_)
    e_ = rs[:, pl.ds(0, 56, 2), :]
    o_ = rs[:, pl.ds(1, 56, 2), :]
    p_ = jnp.concatenate(
        [jnp.zeros((56, 1, 128), _F32), o_[:, 0:55, :]], axis=1)
    pooled = jnp.maximum(jnp.maximum(e_, o_), p_).astype(_BF)
    row = jnp.concatenate(
        [jnp.zeros((56, 1, 128), _BF), pooled, jnp.zeros((56, 7, 128), _BF)],
        axis=1)
    zz = jnp.concatenate(
        [jnp.zeros((1, 64, 128), _BF), row, jnp.zeros((1, 64, 128), _BF)],
        axis=0)
    o_ref[...] = zz.reshape(3712, 128)


def _stem(x, cb, N):
    xn = jnp.transpose(x.astype(_BF), (0, 2, 3, 1))
    xp = jnp.pad(xn, ((0, 0), (3, 3), (3, 3), (0, 0)))
    x2 = xp.reshape(N, 115, 2, 115, 2, 3).transpose(0, 1, 3, 2, 4, 5)
    x2 = x2.reshape(N, 115, 115, 12)
    x2 = jnp.pad(x2, ((0, 0), (0, 1), (0, 5), (0, 4)))
    x2 = x2.reshape(N * _S0, 16)

    wt = cb["w"]
    wp = jnp.pad(wt, ((0, 0), (0, 0), (0, 1), (0, 1)))
    wp = wp.reshape(64, 3, 4, 2, 4, 2).transpose(2, 4, 3, 5, 1, 0)
    wp = wp.reshape(4, 4, 12, 64)
    wp = jnp.pad(wp, ((0, 0), (0, 0), (0, 4), (0, 0)))
    wst = wp.reshape(256, 64).astype(_BF)
    sst, hst = _bn_scale_shift(cb, 64)

    return pl.pallas_call(
        _stem_kernel,
        out_shape=jax.ShapeDtypeStruct((N * 3712, 128), _BF),
        grid=(N,),
        in_specs=[
            pl.BlockSpec((_S0, 16), lambda i: (i, 0)),
            pl.BlockSpec((256, 64), lambda i: (0, 0)),
            pl.BlockSpec((1, 64), lambda i: (0, 0)),
            pl.BlockSpec((1, 64), lambda i: (0, 0)),
        ],
        out_specs=pl.BlockSpec((3712, 128), lambda i: (i, 0)),
        scratch_shapes=[pltpu.VMEM((112, _RW0, 128), _F32),
                        pltpu.VMEM((56, _RW0, 128), _F32)],
        compiler_params=pltpu.CompilerParams(
            dimension_semantics=("arbitrary",),
            vmem_limit_bytes=60 * 1024 * 1024),
    )(x2, wst, sst, hst)


def _trans_kernel(x_ref, w1r, s1r, h1r, w2r, s2r, h2r, w3r, s3r, h3r,
                  wdr, sdr, hdr, o_ref, y1s3, xf, *, B, SA, rowsA, RWA,
                  rowsB, RWB, HA, WA, HB, WB, C1, Cin, Cout):
    mskA = _interior(0, B * SA, SA, RWA, HA, WA)
    xv = x_ref[...]
    x = xv.reshape(B * SA, Cin)

    a1 = jnp.dot(x, w1r[...], preferred_element_type=_F32)
    y1 = jnp.maximum(a1 * s1r[...] + h1r[...], 0.0) * mskA
    y1s3[...] = y1.reshape(B * rowsA, RWA, C1 // 128, 128)

    rA2 = B * rowsA // 2
    rI2 = rowsA // 2
    cA2 = RWA // 2
    pp = {}
    for pr in (0, 1):
        for pc in (0, 1):
            v = y1s3[pl.ds(pr, rA2, 2), pl.ds(pc, cA2, 2), :, :]
            pp[(pr, pc)] = v.reshape(rA2, cA2, C1).astype(_BF)

    M2 = B * HB * WB
    acc = jnp.zeros((M2, C1), _F32)
    for a in range(3):
        for b in range(3):
            plane = pp[(a % 2, b % 2)]
            dr, dc = a // 2, b // 2
            parts = []
            for im in range(B):
                r0 = im * rI2 + dr
                t = plane[r0:r0 + HB, dc:dc + WB, :]
                parts.append(t.reshape(HB * WB, C1))
            tap = jnp.concatenate(parts, axis=0) if B > 1 else parts[0]
            k = a * 3 + b
            acc = acc + jnp.dot(tap, w2r[k * C1:(k + 1) * C1, :],
                                preferred_element_type=_F32)
    y2 = jnp.maximum(acc * s2r[...] + h2r[...], 0.0).astype(_BF)

    xf[...] = xv.astype(_F32).reshape(B * rowsA, RWA, Cin // 128, 128)
    px = (xf[pl.ds(1, rA2, 2), pl.ds(1, cA2, 2), :, :]
          .reshape(rA2, cA2, Cin).astype(_BF))
    parts = []
    for im in range(B):
        t = px[im * rI2:im * rI2 + HB, 0:WB, :]
        parts.append(t.reshape(HB * WB, Cin))
    sub = jnp.concatenate(parts, axis=0) if B > 1 else parts[0]
    ident = (jnp.dot(sub, wdr[...], preferred_element_type=_F32)
             * sdr[...] + hdr[...])

    a3 = jnp.dot(y2, w3r[...], preferred_element_type=_F32) * s3r[...] + h3r[...]
    out = jnp.maximum(a3 + ident, 0.0).astype(_BF)

    imgs = []
    ztop = jnp.zeros((1, RWB, Cout), _BF)
    zbot = jnp.zeros((rowsB - HB - 1, RWB, Cout), _BF)
    zl = jnp.zeros((HB, 1, Cout), _BF)
    zr = jnp.zeros((HB, RWB - WB - 1, Cout), _BF)
    for im in range(B):
        blockv = out[im * HB * WB:(im + 1) * HB * WB].reshape(HB, WB, Cout)
        blockv = jnp.concatenate([zl, blockv, zr], axis=1)
        imgs.append(jnp.concatenate([ztop, blockv, zbot], axis=0))
    zz = jnp.concatenate(imgs, axis=0) if B > 1 else imgs[0]
    o_ref[...] = zz.reshape(B * rowsB * RWB, Cout)


def _transition(x, bp, HA, WA, cin_p, c1_p, cout_p, N):
    rowsA, RWA, B = _PLAN[HA]
    SA = rowsA * RWA
    HB, WB = HA // 2, WA // 2
    rowsB, RWB, _ = _PLAN[HB]
    SB = rowsB * RWB

    w1 = _w_1x1(bp["conv1"]["w"], cin_p, c1_p)
    s1, h1 = _bn_scale_shift(bp["conv1"], c1_p)
    w2 = _w_3x3(bp["conv2"]["w"], c1_p, c1_p)
    s2, h2 = _bn_scale_shift(bp["conv2"], c1_p)
    w3 = _w_1x1(bp["conv3"]["w"], c1_p, cout_p)
    s3, h3 = _bn_scale_shift(bp["conv3"], cout_p)
    wd = _w_1x1(bp["ds"]["w"], cin_p, cout_p)
    sd, hd = _bn_scale_shift(bp["ds"], cout_p)

    return pl.pallas_call(
        functools.partial(_trans_kernel, B=B, SA=SA, rowsA=rowsA, RWA=RWA,
                          rowsB=rowsB, RWB=RWB, HA=HA, WA=WA, HB=HB, WB=WB,
                          C1=c1_p, Cin=cin_p, Cout=cout_p),
        out_shape=jax.ShapeDtypeStruct((N * SB, cout_p), _BF),
        grid=(N // B,),
        in_specs=[
            pl.BlockSpec((B * rowsA, RWA, cin_p), lambda i: (i, 0, 0)),
            pl.BlockSpec((cin_p, c1_p), lambda i: (0, 0)),
            pl.BlockSpec((1, c1_p), lambda i: (0, 0)),
            pl.BlockSpec((1, c1_p), lambda i: (0, 0)),
            pl.BlockSpec((9 * c1_p, c1_p), lambda i: (0, 0)),
            pl.BlockSpec((1, c1_p), lambda i: (0, 0)),
            pl.BlockSpec((1, c1_p), lambda i: (0, 0)),
            pl.BlockSpec((c1_p, cout_p), lambda i: (0, 0)),
            pl.BlockSpec((1, cout_p), lambda i: (0, 0)),
            pl.BlockSpec((1, cout_p), lambda i: (0, 0)),
            pl.BlockSpec((cin_p, cout_p), lambda i: (0, 0)),
            pl.BlockSpec((1, cout_p), lambda i: (0, 0)),
            pl.BlockSpec((1, cout_p), lambda i: (0, 0)),
        ],
        out_specs=pl.BlockSpec((B * SB, cout_p), lambda i: (i, 0)),
        scratch_shapes=[
            pltpu.VMEM((B * rowsA, RWA, c1_p // 128, 128), _F32),
            pltpu.VMEM((B * rowsA, RWA, cin_p // 128, 128), _F32)],
        compiler_params=pltpu.CompilerParams(
            dimension_semantics=("arbitrary",),
            vmem_limit_bytes=60 * 1024 * 1024),
    )(x.reshape(N * rowsA, RWA, cin_p), w1, s1, h1, w2, s2, h2, w3, s3, h3,
      wd, sd, hd)


def _fc_kernel(a_ref, x_ref, w_ref, b_ref, o_ref, *, inv_s):
    t = jnp.dot(a_ref[...], x_ref[...], preferred_element_type=_F32)
    xm = (t * inv_s).astype(_BF)
    o_ref[...] = jnp.dot(xm, w_ref[...], preferred_element_type=_F32) + b_ref[...]


def _avgpool_fc(x, fc_w, fc_b, S, H, W, N):
    C = x.shape[1]
    nc = fc_w.shape[0]
    ncp = _ru(nc, 128)
    amat = jnp.repeat(jnp.eye(N, dtype=_BF), S, axis=1)
    wm = jnp.pad(jnp.transpose(fc_w), ((0, 0), (0, ncp - nc))).astype(_BF)
    bv = jnp.pad(fc_b, (0, ncp - nc)).reshape(1, ncp).astype(_F32)
    out = pl.pallas_call(
        functools.partial(_fc_kernel, inv_s=1.0 / float(H * W)),
        out_shape=jax.ShapeDtypeStruct((N, ncp), _F32),
        grid=(1,),
        in_specs=[
            pl.BlockSpec((N, N * S), lambda i: (0, 0)),
            pl.BlockSpec((N * S, C), lambda i: (0, 0)),
            pl.BlockSpec((C, ncp), lambda i: (0, 0)),
            pl.BlockSpec((1, ncp), lambda i: (0, 0)),
        ],
        out_specs=pl.BlockSpec((N, ncp), lambda i: (0, 0)),
        compiler_params=pltpu.CompilerParams(
            dimension_semantics=("arbitrary",),
            vmem_limit_bytes=60 * 1024 * 1024),
    )(amat, x, wm, bv)
    return out[:, :nc]


_SUF = ("w", "b", "gamma", "beta", "mean", "var")


def kernel(*args):
    x = args[0]
    stem = dict(zip(_SUF, args[1:7]))
    idx = 7
    layers = []
    for nblocks in (3, 4, 6, 3):
        blocks = []
        for b in range(nblocks):
            bp = {}
            for cname in ("conv1", "conv2", "conv3"):
                bp[cname] = dict(zip(_SUF, args[idx:idx + 6]))
                idx += 6
            if b == 0:
                bp["ds"] = dict(zip(_SUF, args[idx:idx + 6]))
                idx += 6
            blocks.append(bp)
        layers.append(blocks)
    fc_w, fc_b = args[idx], args[idx + 1]

    N = x.shape[0]

    h = _stem(x, stem, N)

    cfg = [(56, 128, 128, 256), (28, 256, 128, 512),
           (14, 512, 256, 1024), (7, 1024, 512, 2048)]
    for li, (H, cin_p, c1_p, cout_p) in enumerate(cfg):
        blocks = layers[li]
        if li == 0:
            h = _bottleneck(h, blocks[0]["conv1"], blocks[0]["conv2"],
                            blocks[0]["conv3"], blocks[0]["ds"],
                            H, H, cin_p, c1_p, cout_p, N)
        else:
            h = _transition(h, blocks[0], H * 2, H * 2, cin_p, c1_p, cout_p, N)
        for bp in blocks[1:]:
            h = _bottleneck(h, bp["conv1"], bp["conv2"], bp["conv3"], None,
                            H, H, cout_p, c1_p, cout_p, N)

    rows4, RW4, _ = _PLAN[7]
    return _avgpool_fc(h, fc_w, fc_b, rows4 * RW4, 7, 7, N)
